# Initial kernel scaffold; baseline (speedup 1.0000x reference)
import functools

import jax
import jax.numpy as jnp
from jax import lax
from jax.experimental import pallas as pl
from jax.experimental.pallas import tpu as pltpu
from jax.experimental.pallas import tpu_sc as plsc

N = 10000
E = 320000
ND = 128
IED = 16
ED = 128

NC = 2
NS = 16
NW = NC * NS

GCHUNK = 400
PER_TILE = E // NW
GITERS = PER_TILE // GCHUNK

IW = 80
JROWS = E // IW
TROWS = JROWS // NW
NSLICE = N // NS

BD = 512
ND_BLOCKS = E // BD

BF = 1000
NF_BLOCKS = N // BF


def _silu(x):
    return x * jax.nn.sigmoid(x)


def _node_embed_kernel(species_ref, wa_ref, ba_ref, f_ref):
    f_ref[...] = (
        jnp.dot(species_ref[...], wa_ref[...],
                preferred_element_type=jnp.float32)
        + ba_ref[...]
    )


def _node_embed(species, Wa, ba):
    return pl.pallas_call(
        _node_embed_kernel,
        out_shape=jax.ShapeDtypeStruct((N, ND), jnp.float32),
    )(species, Wa, ba.reshape(1, ND))


def _gather_body(f_hbm, i_hbm, j_hbm, fi_hbm, fj_hbm, idx_v, rows_v, sem):
    wid = lax.axis_index("s") * NC + lax.axis_index("c")
    base = wid * PER_TILE

    @pl.loop(0, GITERS)
    def _(k):
        off = base + k * GCHUNK
        pltpu.sync_copy(i_hbm.at[pl.ds(off, GCHUNK)], idx_v)
        pltpu.async_copy(f_hbm.at[idx_v], rows_v, sem).wait()
        pltpu.sync_copy(rows_v, fi_hbm.at[pl.ds(off, GCHUNK)])
        pltpu.sync_copy(j_hbm.at[pl.ds(off, GCHUNK)], idx_v)
        pltpu.async_copy(f_hbm.at[idx_v], rows_v, sem).wait()
        pltpu.sync_copy(rows_v, fj_hbm.at[pl.ds(off, GCHUNK)])


def _gather_sc(f, i, j):
    mesh = plsc.VectorSubcoreMesh(core_axis_name="c", subcore_axis_name="s")
    fs = jax.ShapeDtypeStruct((E, ND), jnp.float32)
    run = pl.kernel(
        _gather_body,
        out_type=(fs, fs),
        mesh=mesh,
        scratch_types=[
            pltpu.VMEM((GCHUNK,), jnp.int32),
            pltpu.VMEM((GCHUNK, ND), jnp.float32),
            pltpu.SemaphoreType.DMA,
        ],
    )
    return run(f, i, j)


def _edge_mlp_kernel(fi_ref, fj_ref, eattr_ref, evec_ref,
                     wb1_ref, bb1_ref, wb2_ref, bb2_ref,
                     ws1_ref, bs1_ref, ws2_ref, bs2_ref,
                     wv1_ref, bv1_ref, wv2_ref, bv2_ref,
                     ea_ref, s_ref, t0_ref, t1_ref, t2_ref):
    f32 = jnp.float32
    fi = fi_ref[...]
    fj = fj_ref[...]

    eh = _silu(jnp.dot(eattr_ref[...], wb1_ref[...],
                       preferred_element_type=f32) + bb1_ref[...])
    ea = jnp.dot(eh, wb2_ref[...], preferred_element_type=f32) + bb2_ref[...]
    ea_ref[...] = ea

    hs = _silu(
        jnp.dot(fi, ws1_ref[0:ND, :], preferred_element_type=f32)
        + jnp.dot(fj, ws1_ref[ND:2 * ND, :], preferred_element_type=f32)
        + jnp.dot(ea, ws1_ref[2 * ND:, :], preferred_element_type=f32)
        + bs1_ref[...]
    )
    s_ref[...] = (jnp.dot(hs, ws2_ref[...], preferred_element_type=f32)
                  + bs2_ref[...]) * fi

    hv = _silu(
        jnp.dot(fi, wv1_ref[0:ND, :], preferred_element_type=f32)
        + jnp.dot(fj, wv1_ref[ND:2 * ND, :], preferred_element_type=f32)
        + jnp.dot(ea, wv1_ref[2 * ND:, :], preferred_element_type=f32)
        + bv1_ref[...]
    )
    pv = jnp.dot(hv, wv2_ref[...], preferred_element_type=f32) + bv2_ref[...]
    ev = evec_ref[...]
    t0_ref[...] = pv * ev[:, 0:1]
    t1_ref[...] = pv * ev[:, 1:2]
    t2_ref[...] = pv * ev[:, 2:3]


def _edge_mlps(fi, fj, edge_attr, edge_vec,
               Wb1, bb1, Wb2, bb2, Ws1, bs1, Ws2, bs2, Wv1, bv1, Wv2, bv2):
    eb = lambda w: pl.BlockSpec((BD, w), lambda b: (b, 0))
    full = lambda *shape: pl.BlockSpec(shape, lambda b: (0,) * len(shape))
    out_spec = pl.BlockSpec((BD, ND), lambda b: (b, 0))
    fs = jax.ShapeDtypeStruct((E, ND), jnp.float32)
    return pl.pallas_call(
        _edge_mlp_kernel,
        grid=(ND_BLOCKS,),
        in_specs=[
            eb(ND), eb(ND), eb(IED), eb(3),
            full(IED, ED), full(1, ED), full(ED, ED), full(1, ED),
            full(2 * ND + ED, ED), full(1, ED), full(ED, ND), full(1, ND),
            full(2 * ND + ED, ED), full(1, ED), full(ED, ND), full(1, ND),
        ],
        out_specs=[out_spec] * 5,
        out_shape=[fs] * 5,
    )(fi, fj, edge_attr, edge_vec,
      Wb1, bb1.reshape(1, ED), Wb2, bb2.reshape(1, ED),
      Ws1, bs1.reshape(1, ED), Ws2, bs2.reshape(1, ND),
      Wv1, bv1.reshape(1, ED), Wv2, bv2.reshape(1, ND))


def _scatter_body(s_hbm, t0_hbm, t1_hbm, t2_hbm, j2_hbm, z_hbm, out_hbm,
                  idx_v, vals_v, acc, sem):
    cid = lax.axis_index("c")
    sid = lax.axis_index("s")
    tile = cid * NS + sid
    row0 = tile * TROWS
    ebase = tile * PER_TILE

    pltpu.sync_copy(j2_hbm.at[pl.ds(row0, TROWS)], idx_v)

    for p, vals_hbm in enumerate((s_hbm, t0_hbm, t1_hbm, t2_hbm)):
        pltpu.sync_copy(z_hbm.at[pl.ds(sid * NSLICE, NSLICE)],
                        acc.at[pl.ds(sid * NSLICE, NSLICE)])
        plsc.subcore_barrier()

        @pl.loop(0, TROWS)
        def _(q):
            pltpu.sync_copy(vals_hbm.at[pl.ds(ebase + q * IW, IW)], vals_v)
            pltpu.sync_copy(vals_v, acc.at[idx_v.at[q]], add=True)

        plsc.subcore_barrier()
        pltpu.sync_copy(acc.at[pl.ds(sid * NSLICE, NSLICE)],
                        out_hbm.at[p, cid, pl.ds(sid * NSLICE, NSLICE)])
        plsc.subcore_barrier()


def _scatter_sc(s, t0, t1, t2, j):
    mesh = plsc.VectorSubcoreMesh(core_axis_name="c", subcore_axis_name="s")
    j2 = j.reshape(JROWS, IW)
    zeros = jnp.zeros((N, ND), jnp.float32)
    run = pl.kernel(
        _scatter_body,
        out_type=jax.ShapeDtypeStruct((4, NC, N, ND), jnp.float32),
        mesh=mesh,
        scratch_types=[
            pltpu.VMEM((TROWS, IW), jnp.int32),
            pltpu.VMEM((IW, ND), jnp.float32),
            pltpu.VMEM_SHARED((N, ND), jnp.float32),
            pltpu.SemaphoreType.DMA,
        ],
    )
    return run(s, t0, t1, t2, j2, zeros)


def _final_kernel(f_ref, p_ref, wh1_ref, bh1_ref, wh2_ref, bh2_ref,
                  h0_ref, v0_ref, v1_ref, v2_ref):
    f32 = jnp.float32
    agg = p_ref[0, 0] + p_ref[0, 1]
    hh = _silu(
        jnp.dot(f_ref[...], wh1_ref[0:ND, :], preferred_element_type=f32)
        + jnp.dot(agg, wh1_ref[ND:, :], preferred_element_type=f32)
        + bh1_ref[...]
    )
    h0_ref[...] = jnp.dot(hh, wh2_ref[...], preferred_element_type=f32) \
        + bh2_ref[...]
    v0_ref[...] = p_ref[1, 0] + p_ref[1, 1]
    v1_ref[...] = p_ref[2, 0] + p_ref[2, 1]
    v2_ref[...] = p_ref[3, 0] + p_ref[3, 1]


def _final_tc(f, partials, Wh1, bh1, Wh2, bh2):
    nb = lambda: pl.BlockSpec((BF, ND), lambda b: (b, 0))
    full = lambda *shape: pl.BlockSpec(shape, lambda b: (0,) * len(shape))
    fs = jax.ShapeDtypeStruct((N, ND), jnp.float32)
    return pl.pallas_call(
        _final_kernel,
        grid=(NF_BLOCKS,),
        in_specs=[
            nb(),
            pl.BlockSpec((4, NC, BF, ND), lambda b: (0, 0, b, 0)),
            full(2 * ND, ED), full(1, ED), full(ED, ND), full(1, ND),
        ],
        out_specs=[nb()] * 4,
        out_shape=[fs] * 4,
    )(f, partials, Wh1, bh1.reshape(1, ED), Wh2, bh2.reshape(1, ND))


def kernel(species, edge_index, edge_attr, edge_vec,
           Wa, ba, Wb1, bb1, Wb2, bb2,
           Ws1, bs1, Ws2, bs2,
           Wh1, bh1, Wh2, bh2,
           Wv1, bv1, Wv2, bv2):
    i = edge_index[0]
    j = edge_index[1]

    f = _node_embed(species, Wa, ba)
    fi, fj = _gather_sc(f, i, j)
    ea, s, t0, t1, t2 = _edge_mlps(
        fi, fj, edge_attr, edge_vec,
        Wb1, bb1, Wb2, bb2, Ws1, bs1, Ws2, bs2, Wv1, bv1, Wv2, bv2)
    partials = _scatter_sc(s, t0, t1, t2, j)
    h0, v00, v01, v02 = _final_tc(f, partials, Wh1, bh1, Wh2, bh2)
    v0 = jnp.stack([v00, v01, v02], axis=-1)
    return (h0, v0, ea)


# speedup vs baseline: 16.5897x; 16.5897x over previous
import functools

import jax
import jax.numpy as jnp
from jax import lax
from jax.experimental import pallas as pl
from jax.experimental.pallas import tpu as pltpu
from jax.experimental.pallas import tpu_sc as plsc

N = 10000
E = 320000
ND = 128
IED = 16
ED = 128

NC = 2
NS = 16
NW = NC * NS

E_PAD = 327680
N_ACC = 10240

GCHUNK = 512
PER_TILE = E_PAD // NW
GITERS = PER_TILE // GCHUNK

IW = 128
JROWS = E_PAD // IW
TROWS = JROWS // NW
NSLICE = N_ACC // NS

BD = 512
ND_BLOCKS = E_PAD // BD
ND_REAL_BLOCKS = E // BD

BF = 1000
NF_BLOCKS = N // BF


def _silu(x):
    return x * jax.nn.sigmoid(x)


def _node_embed_kernel(species_ref, wa_ref, ba_ref, f_ref):
    f_ref[...] = (
        jnp.dot(species_ref[...], wa_ref[...],
                preferred_element_type=jnp.float32)
        + ba_ref[...]
    )


def _node_embed(species, Wa, ba):
    return pl.pallas_call(
        _node_embed_kernel,
        out_shape=jax.ShapeDtypeStruct((N, ND), jnp.float32),
    )(species, Wa, ba.reshape(1, ND))


def _gather_body(f_hbm, i_hbm, j_hbm, fi_hbm, fj_hbm, idx_v, rows_v, sem):
    wid = lax.axis_index("s") * NC + lax.axis_index("c")
    base = wid * PER_TILE

    @pl.loop(0, GITERS)
    def _(k):
        off = base + k * GCHUNK
        pltpu.sync_copy(i_hbm.at[pl.ds(off, GCHUNK)], idx_v)
        pltpu.async_copy(f_hbm.at[idx_v], rows_v, sem).wait()
        pltpu.sync_copy(rows_v, fi_hbm.at[pl.ds(off, GCHUNK)])
        pltpu.sync_copy(j_hbm.at[pl.ds(off, GCHUNK)], idx_v)
        pltpu.async_copy(f_hbm.at[idx_v], rows_v, sem).wait()
        pltpu.sync_copy(rows_v, fj_hbm.at[pl.ds(off, GCHUNK)])


def _gather_sc(f, i, j):
    mesh = plsc.VectorSubcoreMesh(core_axis_name="c", subcore_axis_name="s")
    fs = jax.ShapeDtypeStruct((E_PAD, ND), jnp.float32)
    run = pl.kernel(
        _gather_body,
        out_type=(fs, fs),
        mesh=mesh,
        scratch_types=[
            pltpu.VMEM((GCHUNK,), jnp.int32),
            pltpu.VMEM((GCHUNK, ND), jnp.float32),
            pltpu.SemaphoreType.DMA,
        ],
    )
    return run(f, i, j)


def _edge_mlp_kernel(fi_ref, fj_ref, eattr_ref, evec_ref,
                     wb1_ref, bb1_ref, wb2_ref, bb2_ref,
                     ws1_ref, bs1_ref, ws2_ref, bs2_ref,
                     wv1_ref, bv1_ref, wv2_ref, bv2_ref,
                     ea_ref, s_ref, t0_ref, t1_ref, t2_ref):
    f32 = jnp.float32
    fi = fi_ref[...]
    fj = fj_ref[...]

    eh = _silu(jnp.dot(eattr_ref[...], wb1_ref[...],
                       preferred_element_type=f32) + bb1_ref[...])
    ea = jnp.dot(eh, wb2_ref[...], preferred_element_type=f32) + bb2_ref[...]
    ea_ref[...] = ea

    hs = _silu(
        jnp.dot(fi, ws1_ref[0:ND, :], preferred_element_type=f32)
        + jnp.dot(fj, ws1_ref[ND:2 * ND, :], preferred_element_type=f32)
        + jnp.dot(ea, ws1_ref[2 * ND:, :], preferred_element_type=f32)
        + bs1_ref[...]
    )
    s_ref[...] = (jnp.dot(hs, ws2_ref[...], preferred_element_type=f32)
                  + bs2_ref[...]) * fi

    hv = _silu(
        jnp.dot(fi, wv1_ref[0:ND, :], preferred_element_type=f32)
        + jnp.dot(fj, wv1_ref[ND:2 * ND, :], preferred_element_type=f32)
        + jnp.dot(ea, wv1_ref[2 * ND:, :], preferred_element_type=f32)
        + bv1_ref[...]
    )
    pv = jnp.dot(hv, wv2_ref[...], preferred_element_type=f32) + bv2_ref[...]
    ev = evec_ref[...]
    t0_ref[...] = pv * ev[:, 0:1]
    t1_ref[...] = pv * ev[:, 1:2]
    t2_ref[...] = pv * ev[:, 2:3]

    @pl.when(pl.program_id(0) >= ND_REAL_BLOCKS)
    def _():
        s_ref[...] = jnp.zeros_like(s_ref)


def _edge_mlps(fi, fj, edge_attr, edge_vec,
               Wb1, bb1, Wb2, bb2, Ws1, bs1, Ws2, bs2, Wv1, bv1, Wv2, bv2):
    eb = lambda w: pl.BlockSpec((BD, w), lambda b: (b, 0))
    full = lambda *shape: pl.BlockSpec(shape, lambda b: (0,) * len(shape))
    out_spec = pl.BlockSpec((BD, ND), lambda b: (b, 0))
    fs = jax.ShapeDtypeStruct((E_PAD, ND), jnp.float32)
    return pl.pallas_call(
        _edge_mlp_kernel,
        grid=(ND_BLOCKS,),
        in_specs=[
            eb(ND), eb(ND), eb(IED), eb(3),
            full(IED, ED), full(1, ED), full(ED, ED), full(1, ED),
            full(2 * ND + ED, ED), full(1, ED), full(ED, ND), full(1, ND),
            full(2 * ND + ED, ED), full(1, ED), full(ED, ND), full(1, ND),
        ],
        out_specs=[out_spec] * 5,
        out_shape=[fs] * 5,
    )(fi, fj, edge_attr, edge_vec,
      Wb1, bb1.reshape(1, ED), Wb2, bb2.reshape(1, ED),
      Ws1, bs1.reshape(1, ED), Ws2, bs2.reshape(1, ND),
      Wv1, bv1.reshape(1, ED), Wv2, bv2.reshape(1, ND))


def _scatter_body(s_hbm, t0_hbm, t1_hbm, t2_hbm, j2_hbm, z_hbm, out_hbm,
                  idx_v, vals_v, acc, sem):
    cid = lax.axis_index("c")
    sid = lax.axis_index("s")
    tile = cid * NS + sid
    row0 = tile * TROWS
    ebase = tile * PER_TILE

    pltpu.sync_copy(j2_hbm.at[pl.ds(row0, TROWS)], idx_v)

    for p, vals_hbm in enumerate((s_hbm, t0_hbm, t1_hbm, t2_hbm)):
        pltpu.sync_copy(z_hbm.at[pl.ds(sid * NSLICE, NSLICE)],
                        acc.at[pl.ds(sid * NSLICE, NSLICE)])
        plsc.subcore_barrier()

        @pl.loop(0, TROWS)
        def _(q):
            pltpu.sync_copy(vals_hbm.at[pl.ds(ebase + q * IW, IW)], vals_v)
            pltpu.sync_copy(vals_v, acc.at[idx_v.at[q]], add=True)

        plsc.subcore_barrier()
        pltpu.sync_copy(acc.at[pl.ds(sid * NSLICE, NSLICE)],
                        out_hbm.at[p, cid, pl.ds(sid * NSLICE, NSLICE)])
        plsc.subcore_barrier()


def _scatter_sc(s, t0, t1, t2, j):
    mesh = plsc.VectorSubcoreMesh(core_axis_name="c", subcore_axis_name="s")
    j2 = j.reshape(JROWS, IW)
    zeros = jnp.zeros((N_ACC, ND), jnp.float32)
    run = pl.kernel(
        _scatter_body,
        out_type=jax.ShapeDtypeStruct((4, NC, N_ACC, ND), jnp.float32),
        mesh=mesh,
        scratch_types=[
            pltpu.VMEM((TROWS, IW), jnp.int32),
            pltpu.VMEM((IW, ND), jnp.float32),
            pltpu.VMEM_SHARED((N_ACC, ND), jnp.float32),
            pltpu.SemaphoreType.DMA,
        ],
    )
    return run(s, t0, t1, t2, j2, zeros)


def _final_kernel(f_ref, p_ref, wh1_ref, bh1_ref, wh2_ref, bh2_ref,
                  h0_ref, v0_ref, v1_ref, v2_ref):
    f32 = jnp.float32
    agg = p_ref[0, 0] + p_ref[0, 1]
    hh = _silu(
        jnp.dot(f_ref[...], wh1_ref[0:ND, :], preferred_element_type=f32)
        + jnp.dot(agg, wh1_ref[ND:, :], preferred_element_type=f32)
        + bh1_ref[...]
    )
    h0_ref[...] = jnp.dot(hh, wh2_ref[...], preferred_element_type=f32) \
        + bh2_ref[...]
    v0_ref[...] = p_ref[1, 0] + p_ref[1, 1]
    v1_ref[...] = p_ref[2, 0] + p_ref[2, 1]
    v2_ref[...] = p_ref[3, 0] + p_ref[3, 1]


def _final_tc(f, partials, Wh1, bh1, Wh2, bh2):
    nb = lambda: pl.BlockSpec((BF, ND), lambda b: (b, 0))
    full = lambda *shape: pl.BlockSpec(shape, lambda b: (0,) * len(shape))
    fs = jax.ShapeDtypeStruct((N, ND), jnp.float32)
    return pl.pallas_call(
        _final_kernel,
        grid=(NF_BLOCKS,),
        in_specs=[
            nb(),
            pl.BlockSpec((4, NC, BF, ND), lambda b: (0, 0, b, 0)),
            full(2 * ND, ED), full(1, ED), full(ED, ND), full(1, ND),
        ],
        out_specs=[nb()] * 4,
        out_shape=[fs] * 4,
    )(f, partials, Wh1, bh1.reshape(1, ED), Wh2, bh2.reshape(1, ND))


def kernel(species, edge_index, edge_attr, edge_vec,
           Wa, ba, Wb1, bb1, Wb2, bb2,
           Ws1, bs1, Ws2, bs2,
           Wh1, bh1, Wh2, bh2,
           Wv1, bv1, Wv2, bv2):
    npad = E_PAD - E
    pad_idx = (jnp.arange(npad, dtype=jnp.int32) * 8) % N
    i = jnp.concatenate([edge_index[0], pad_idx])
    j = jnp.concatenate([edge_index[1], pad_idx])
    edge_attr_p = jnp.pad(edge_attr, ((0, npad), (0, 0)))
    edge_vec_p = jnp.pad(edge_vec, ((0, npad), (0, 0)))

    f = _node_embed(species, Wa, ba)
    fi, fj = _gather_sc(f, i, j)
    ea_p, s, t0, t1, t2 = _edge_mlps(
        fi, fj, edge_attr_p, edge_vec_p,
        Wb1, bb1, Wb2, bb2, Ws1, bs1, Ws2, bs2, Wv1, bv1, Wv2, bv2)
    partials = _scatter_sc(s, t0, t1, t2, j)
    h0, v00, v01, v02 = _final_tc(f, partials, Wh1, bh1, Wh2, bh2)
    v0 = jnp.stack([v00, v01, v02], axis=-1)
    return (h0, v0, ea_p[:E])


# speedup vs baseline: 18.7988x; 1.1332x over previous
import functools

import jax
import jax.numpy as jnp
from jax import lax
from jax.experimental import pallas as pl
from jax.experimental.pallas import tpu as pltpu
from jax.experimental.pallas import tpu_sc as plsc

N = 10000
E = 320000
ND = 128
IED = 16
ED = 128

NC = 2
NS = 16
NW = NC * NS

E_PAD = 327680
N_ACC = 10240

GCHUNK = 320
PER_TILE = E_PAD // NW
GITERS = PER_TILE // GCHUNK

IW = 128
JROWS = E_PAD // IW
TROWS = JROWS // NW
NSLICE = N_ACC // NS

BD = 512
ND_BLOCKS = E_PAD // BD
ND_REAL_BLOCKS = E // BD

BF = 1000
NF_BLOCKS = N // BF


def _silu(x):
    return x * jax.nn.sigmoid(x)


def _node_embed_kernel(species_ref, wa_ref, ba_ref, f_ref):
    f_ref[...] = (
        jnp.dot(species_ref[...], wa_ref[...],
                preferred_element_type=jnp.float32)
        + ba_ref[...]
    )


def _node_embed(species, Wa, ba):
    return pl.pallas_call(
        _node_embed_kernel,
        out_shape=jax.ShapeDtypeStruct((N, ND), jnp.float32),
    )(species, Wa, ba.reshape(1, ND))


def _gather_body(f_hbm, i_hbm, j_hbm, fi_hbm, fj_hbm,
                 idxi_v, idxj_v, rowsi_v, rowsj_v,
                 gsem, wsem_i, wsem_j):
    wid = lax.axis_index("s") * NC + lax.axis_index("c")
    base = wid * PER_TILE

    @pl.loop(0, GITERS)
    def _(k):
        off = base + k * GCHUNK
        pltpu.sync_copy(i_hbm.at[pl.ds(off, GCHUNK)], idxi_v)
        pltpu.sync_copy(j_hbm.at[pl.ds(off, GCHUNK)], idxj_v)

        @pl.when(k > 0)
        def _():
            pltpu.make_async_copy(rowsi_v, fi_hbm.at[pl.ds(off, GCHUNK)],
                                  wsem_i).wait()
        pltpu.async_copy(f_hbm.at[idxi_v], rowsi_v, gsem).wait()
        pltpu.async_copy(rowsi_v, fi_hbm.at[pl.ds(off, GCHUNK)], wsem_i)

        @pl.when(k > 0)
        def _():
            pltpu.make_async_copy(rowsj_v, fj_hbm.at[pl.ds(off, GCHUNK)],
                                  wsem_j).wait()
        pltpu.async_copy(f_hbm.at[idxj_v], rowsj_v, gsem).wait()
        pltpu.async_copy(rowsj_v, fj_hbm.at[pl.ds(off, GCHUNK)], wsem_j)

    pltpu.make_async_copy(rowsi_v, fi_hbm.at[pl.ds(base, GCHUNK)],
                          wsem_i).wait()
    pltpu.make_async_copy(rowsj_v, fj_hbm.at[pl.ds(base, GCHUNK)],
                          wsem_j).wait()


def _gather_sc(f, i, j):
    mesh = plsc.VectorSubcoreMesh(core_axis_name="c", subcore_axis_name="s")
    fs = jax.ShapeDtypeStruct((E_PAD, ND), jnp.float32)
    run = pl.kernel(
        _gather_body,
        out_type=(fs, fs),
        mesh=mesh,
        scratch_types=[
            pltpu.VMEM((GCHUNK,), jnp.int32),
            pltpu.VMEM((GCHUNK,), jnp.int32),
            pltpu.VMEM((GCHUNK, ND), jnp.float32),
            pltpu.VMEM((GCHUNK, ND), jnp.float32),
            pltpu.SemaphoreType.DMA,
            pltpu.SemaphoreType.DMA,
            pltpu.SemaphoreType.DMA,
        ],
    )
    return run(f, i, j)


def _edge_mlp_kernel(fi_ref, fj_ref, eattr_ref, evec_ref,
                     wb1_ref, bb1_ref, wb2_ref, bb2_ref,
                     ws1_ref, bs1_ref, ws2_ref, bs2_ref,
                     wv1_ref, bv1_ref, wv2_ref, bv2_ref,
                     ea_ref, s_ref, t0_ref, t1_ref, t2_ref):
    f32 = jnp.float32
    fi = fi_ref[...]
    fj = fj_ref[...]

    eh = _silu(jnp.dot(eattr_ref[...], wb1_ref[...],
                       preferred_element_type=f32) + bb1_ref[...])
    ea = jnp.dot(eh, wb2_ref[...], preferred_element_type=f32) + bb2_ref[...]
    ea_ref[...] = ea

    hs = _silu(
        jnp.dot(fi, ws1_ref[0:ND, :], preferred_element_type=f32)
        + jnp.dot(fj, ws1_ref[ND:2 * ND, :], preferred_element_type=f32)
        + jnp.dot(ea, ws1_ref[2 * ND:, :], preferred_element_type=f32)
        + bs1_ref[...]
    )
    s_ref[...] = (jnp.dot(hs, ws2_ref[...], preferred_element_type=f32)
                  + bs2_ref[...]) * fi

    hv = _silu(
        jnp.dot(fi, wv1_ref[0:ND, :], preferred_element_type=f32)
        + jnp.dot(fj, wv1_ref[ND:2 * ND, :], preferred_element_type=f32)
        + jnp.dot(ea, wv1_ref[2 * ND:, :], preferred_element_type=f32)
        + bv1_ref[...]
    )
    pv = jnp.dot(hv, wv2_ref[...], preferred_element_type=f32) + bv2_ref[...]
    ev = evec_ref[...]
    t0_ref[...] = pv * ev[:, 0:1]
    t1_ref[...] = pv * ev[:, 1:2]
    t2_ref[...] = pv * ev[:, 2:3]

    @pl.when(pl.program_id(0) >= ND_REAL_BLOCKS)
    def _():
        s_ref[...] = jnp.zeros_like(s_ref)


def _edge_mlps(fi, fj, edge_attr, edge_vec,
               Wb1, bb1, Wb2, bb2, Ws1, bs1, Ws2, bs2, Wv1, bv1, Wv2, bv2):
    eb = lambda w: pl.BlockSpec((BD, w), lambda b: (b, 0))
    full = lambda *shape: pl.BlockSpec(shape, lambda b: (0,) * len(shape))
    out_spec = pl.BlockSpec((BD, ND), lambda b: (b, 0))
    fs = jax.ShapeDtypeStruct((E_PAD, ND), jnp.float32)
    return pl.pallas_call(
        _edge_mlp_kernel,
        grid=(ND_BLOCKS,),
        in_specs=[
            eb(ND), eb(ND), eb(IED), eb(3),
            full(IED, ED), full(1, ED), full(ED, ED), full(1, ED),
            full(2 * ND + ED, ED), full(1, ED), full(ED, ND), full(1, ND),
            full(2 * ND + ED, ED), full(1, ED), full(ED, ND), full(1, ND),
        ],
        out_specs=[out_spec] * 5,
        out_shape=[fs] * 5,
    )(fi, fj, edge_attr, edge_vec,
      Wb1, bb1.reshape(1, ED), Wb2, bb2.reshape(1, ED),
      Ws1, bs1.reshape(1, ED), Ws2, bs2.reshape(1, ND),
      Wv1, bv1.reshape(1, ED), Wv2, bv2.reshape(1, ND))


def _scatter_body(s_hbm, t0_hbm, t1_hbm, t2_hbm, j2_hbm, z_hbm, out_hbm,
                  idx_v, vals0_v, vals1_v, acc, dsem0, dsem1):
    cid = lax.axis_index("c")
    sid = lax.axis_index("s")
    tile = cid * NS + sid
    row0 = tile * TROWS
    ebase = tile * PER_TILE

    pltpu.sync_copy(j2_hbm.at[pl.ds(row0, TROWS)], idx_v)

    for p, vals_hbm in enumerate((s_hbm, t0_hbm, t1_hbm, t2_hbm)):
        pltpu.sync_copy(z_hbm.at[pl.ds(sid * NSLICE, NSLICE)],
                        acc.at[pl.ds(sid * NSLICE, NSLICE)])
        plsc.subcore_barrier()

        pltpu.async_copy(vals_hbm.at[pl.ds(ebase, IW)], vals0_v, dsem0)

        @pl.loop(0, TROWS // 2)
        def _(g):
            q0 = 2 * g
            pltpu.async_copy(vals_hbm.at[pl.ds(ebase + (q0 + 1) * IW, IW)],
                             vals1_v, dsem1)
            pltpu.make_async_copy(vals_hbm.at[pl.ds(ebase, IW)], vals0_v,
                                  dsem0).wait()
            pltpu.sync_copy(vals0_v, acc.at[idx_v.at[q0]], add=True)

            @pl.when(g < TROWS // 2 - 1)
            def _():
                pltpu.async_copy(
                    vals_hbm.at[pl.ds(ebase + (q0 + 2) * IW, IW)],
                    vals0_v, dsem0)
            pltpu.make_async_copy(vals_hbm.at[pl.ds(ebase, IW)], vals1_v,
                                  dsem1).wait()
            pltpu.sync_copy(vals1_v, acc.at[idx_v.at[q0 + 1]], add=True)

        plsc.subcore_barrier()
        pltpu.sync_copy(acc.at[pl.ds(sid * NSLICE, NSLICE)],
                        out_hbm.at[p, cid, pl.ds(sid * NSLICE, NSLICE)])
        plsc.subcore_barrier()


def _scatter_sc(s, t0, t1, t2, j):
    mesh = plsc.VectorSubcoreMesh(core_axis_name="c", subcore_axis_name="s")
    j2 = j.reshape(JROWS, IW)
    zeros = jnp.zeros((N_ACC, ND), jnp.float32)
    run = pl.kernel(
        _scatter_body,
        out_type=jax.ShapeDtypeStruct((4, NC, N_ACC, ND), jnp.float32),
        mesh=mesh,
        scratch_types=[
            pltpu.VMEM((TROWS, IW), jnp.int32),
            pltpu.VMEM((IW, ND), jnp.float32),
            pltpu.VMEM((IW, ND), jnp.float32),
            pltpu.VMEM_SHARED((N_ACC, ND), jnp.float32),
            pltpu.SemaphoreType.DMA,
            pltpu.SemaphoreType.DMA,
        ],
    )
    return run(s, t0, t1, t2, j2, zeros)


def _final_kernel(f_ref, p_ref, wh1_ref, bh1_ref, wh2_ref, bh2_ref,
                  h0_ref, v0_ref, v1_ref, v2_ref):
    f32 = jnp.float32
    agg = p_ref[0, 0] + p_ref[0, 1]
    hh = _silu(
        jnp.dot(f_ref[...], wh1_ref[0:ND, :], preferred_element_type=f32)
        + jnp.dot(agg, wh1_ref[ND:, :], preferred_element_type=f32)
        + bh1_ref[...]
    )
    h0_ref[...] = jnp.dot(hh, wh2_ref[...], preferred_element_type=f32) \
        + bh2_ref[...]
    v0_ref[...] = p_ref[1, 0] + p_ref[1, 1]
    v1_ref[...] = p_ref[2, 0] + p_ref[2, 1]
    v2_ref[...] = p_ref[3, 0] + p_ref[3, 1]


def _final_tc(f, partials, Wh1, bh1, Wh2, bh2):
    nb = lambda: pl.BlockSpec((BF, ND), lambda b: (b, 0))
    full = lambda *shape: pl.BlockSpec(shape, lambda b: (0,) * len(shape))
    fs = jax.ShapeDtypeStruct((N, ND), jnp.float32)
    return pl.pallas_call(
        _final_kernel,
        grid=(NF_BLOCKS,),
        in_specs=[
            nb(),
            pl.BlockSpec((4, NC, BF, ND), lambda b: (0, 0, b, 0)),
            full(2 * ND, ED), full(1, ED), full(ED, ND), full(1, ND),
        ],
        out_specs=[nb()] * 4,
        out_shape=[fs] * 4,
    )(f, partials, Wh1, bh1.reshape(1, ED), Wh2, bh2.reshape(1, ND))


def kernel(species, edge_index, edge_attr, edge_vec,
           Wa, ba, Wb1, bb1, Wb2, bb2,
           Ws1, bs1, Ws2, bs2,
           Wh1, bh1, Wh2, bh2,
           Wv1, bv1, Wv2, bv2):
    npad = E_PAD - E
    pad_idx = (jnp.arange(npad, dtype=jnp.int32) * 8) % N
    i = jnp.concatenate([edge_index[0], pad_idx])
    j = jnp.concatenate([edge_index[1], pad_idx])
    edge_attr_p = jnp.pad(edge_attr, ((0, npad), (0, 0)))
    edge_vec_p = jnp.pad(edge_vec, ((0, npad), (0, 0)))

    f = _node_embed(species, Wa, ba)
    fi, fj = _gather_sc(f, i, j)
    ea_p, s, t0, t1, t2 = _edge_mlps(
        fi, fj, edge_attr_p, edge_vec_p,
        Wb1, bb1, Wb2, bb2, Ws1, bs1, Ws2, bs2, Wv1, bv1, Wv2, bv2)
    partials = _scatter_sc(s, t0, t1, t2, j)
    h0, v00, v01, v02 = _final_tc(f, partials, Wh1, bh1, Wh2, bh2)
    v0 = jnp.stack([v00, v01, v02], axis=-1)
    return (h0, v0, ea_p[:E])


# speedup vs baseline: 19.1360x; 1.0179x over previous
import functools

import jax
import jax.numpy as jnp
from jax import lax
from jax.experimental import pallas as pl
from jax.experimental.pallas import tpu as pltpu
from jax.experimental.pallas import tpu_sc as plsc

N = 10000
E = 320000
ND = 128
IED = 16
ED = 128

NC = 2
NS = 16
NW = NC * NS

E_PAD = 327680
N_ACC = 10240

GCHUNK = 320
PER_TILE = E_PAD // NW
GITERS = PER_TILE // GCHUNK

IW = 128
JROWS = E_PAD // IW
TROWS = JROWS // NW
NSLICE = N_ACC // NS

BD = 512
ND_BLOCKS = E_PAD // BD
ND_REAL_BLOCKS = E // BD

BF = 1000
NF_BLOCKS = N // BF


def _silu(x):
    return x * jax.nn.sigmoid(x)


def _node_embed_kernel(species_ref, wa_ref, ba_ref, f_ref):
    f_ref[...] = (
        jnp.dot(species_ref[...], wa_ref[...],
                preferred_element_type=jnp.float32)
        + ba_ref[...]
    )


def _node_embed(species, Wa, ba):
    return pl.pallas_call(
        _node_embed_kernel,
        out_shape=jax.ShapeDtypeStruct((N, ND), jnp.float32),
    )(species, Wa, ba.reshape(1, ND))


def _gather_body(f_hbm, i_hbm, j_hbm, fi_hbm, fj_hbm,
                 idxi_v, idxj_v, rowsi_v, rowsj_v,
                 gsem, wsem_i, wsem_j):
    wid = lax.axis_index("s") * NC + lax.axis_index("c")
    base = wid * PER_TILE

    @pl.loop(0, GITERS)
    def _(k):
        off = base + k * GCHUNK
        pltpu.sync_copy(i_hbm.at[pl.ds(off, GCHUNK)], idxi_v)
        pltpu.sync_copy(j_hbm.at[pl.ds(off, GCHUNK)], idxj_v)

        @pl.when(k > 0)
        def _():
            pltpu.make_async_copy(rowsi_v, fi_hbm.at[pl.ds(off, GCHUNK)],
                                  wsem_i).wait()
        pltpu.async_copy(f_hbm.at[idxi_v], rowsi_v, gsem).wait()
        pltpu.async_copy(rowsi_v, fi_hbm.at[pl.ds(off, GCHUNK)], wsem_i)

        @pl.when(k > 0)
        def _():
            pltpu.make_async_copy(rowsj_v, fj_hbm.at[pl.ds(off, GCHUNK)],
                                  wsem_j).wait()
        pltpu.async_copy(f_hbm.at[idxj_v], rowsj_v, gsem).wait()
        pltpu.async_copy(rowsj_v, fj_hbm.at[pl.ds(off, GCHUNK)], wsem_j)

    pltpu.make_async_copy(rowsi_v, fi_hbm.at[pl.ds(base, GCHUNK)],
                          wsem_i).wait()
    pltpu.make_async_copy(rowsj_v, fj_hbm.at[pl.ds(base, GCHUNK)],
                          wsem_j).wait()


def _gather_sc(f, i, j):
    mesh = plsc.VectorSubcoreMesh(core_axis_name="c", subcore_axis_name="s")
    fs = jax.ShapeDtypeStruct((E_PAD, ND), jnp.float32)
    run = pl.kernel(
        _gather_body,
        out_type=(fs, fs),
        mesh=mesh,
        scratch_types=[
            pltpu.VMEM((GCHUNK,), jnp.int32),
            pltpu.VMEM((GCHUNK,), jnp.int32),
            pltpu.VMEM((GCHUNK, ND), jnp.float32),
            pltpu.VMEM((GCHUNK, ND), jnp.float32),
            pltpu.SemaphoreType.DMA,
            pltpu.SemaphoreType.DMA,
            pltpu.SemaphoreType.DMA,
        ],
    )
    return run(f, i, j)


def _edge_mlp_kernel(fi_ref, fj_ref, eattr_ref, evec_ref,
                     wb1_ref, bb1_ref, wb2_ref, bb2_ref,
                     w1_ref, b1_ref, w2_ref, b2_ref,
                     ea_ref, s_ref, t0_ref, t1_ref, t2_ref):
    f32 = jnp.float32
    bf16 = jnp.bfloat16
    fi32 = fi_ref[...]
    fi = fi32.astype(bf16)
    fj = fj_ref[...].astype(bf16)

    eh = _silu(jnp.dot(eattr_ref[...].astype(bf16), wb1_ref[...],
                       preferred_element_type=f32) + bb1_ref[...])
    ea = jnp.dot(eh.astype(bf16), wb2_ref[...],
                 preferred_element_type=f32) + bb2_ref[...]
    ea_ref[...] = ea

    h = _silu(
        jnp.dot(fi, w1_ref[0:ND, :], preferred_element_type=f32)
        + jnp.dot(fj, w1_ref[ND:2 * ND, :], preferred_element_type=f32)
        + jnp.dot(ea.astype(bf16), w1_ref[2 * ND:, :],
                  preferred_element_type=f32)
        + b1_ref[...]
    ).astype(bf16)
    sp = jnp.dot(h, w2_ref[...], preferred_element_type=f32) + b2_ref[...]

    s = sp[:, 0:ND] * fi32
    pv = sp[:, ND:]
    s_ref[...] = jnp.where(pl.program_id(0) < ND_REAL_BLOCKS, s, 0.0)
    ev = evec_ref[...]
    t0_ref[...] = pv * ev[:, 0:1]
    t1_ref[...] = pv * ev[:, 1:2]
    t2_ref[...] = pv * ev[:, 2:3]


def _edge_mlps(fi, fj, edge_attr, edge_vec,
               Wb1, bb1, Wb2, bb2, Ws1, bs1, Ws2, bs2, Wv1, bv1, Wv2, bv2):
    bf16 = jnp.bfloat16
    w1 = jnp.concatenate([Ws1, Wv1], axis=1).astype(bf16)
    b1 = jnp.concatenate([bs1, bv1]).reshape(1, 2 * ND)
    w2 = jnp.zeros((2 * ND, 2 * ND), jnp.float32)
    w2 = w2.at[0:ND, 0:ND].set(Ws2).at[ND:, ND:].set(Wv2).astype(bf16)
    b2 = jnp.concatenate([bs2, bv2]).reshape(1, 2 * ND)

    eb = lambda w: pl.BlockSpec((BD, w), lambda b: (b, 0))
    full = lambda *shape: pl.BlockSpec(shape, lambda b: (0,) * len(shape))
    out_spec = pl.BlockSpec((BD, ND), lambda b: (b, 0))
    fs = jax.ShapeDtypeStruct((E_PAD, ND), jnp.float32)
    return pl.pallas_call(
        _edge_mlp_kernel,
        grid=(ND_BLOCKS,),
        in_specs=[
            eb(ND), eb(ND), eb(IED), eb(3),
            full(IED, ED), full(1, ED), full(ED, ED), full(1, ED),
            full(2 * ND + ED, 2 * ND), full(1, 2 * ND),
            full(2 * ND, 2 * ND), full(1, 2 * ND),
        ],
        out_specs=[out_spec] * 5,
        out_shape=[fs] * 5,
    )(fi, fj, edge_attr, edge_vec,
      Wb1.astype(bf16), bb1.reshape(1, ED), Wb2.astype(bf16),
      bb2.reshape(1, ED), w1, b1, w2, b2)


def _scatter_body(s_hbm, t0_hbm, t1_hbm, t2_hbm, j2_hbm, z_hbm, out_hbm,
                  idx_v, vals0_v, vals1_v, acc, dsem0, dsem1):
    cid = lax.axis_index("c")
    sid = lax.axis_index("s")
    tile = cid * NS + sid
    row0 = tile * TROWS
    ebase = tile * PER_TILE

    pltpu.sync_copy(j2_hbm.at[pl.ds(row0, TROWS)], idx_v)

    for p, vals_hbm in enumerate((s_hbm, t0_hbm, t1_hbm, t2_hbm)):
        pltpu.sync_copy(z_hbm.at[pl.ds(sid * NSLICE, NSLICE)],
                        acc.at[pl.ds(sid * NSLICE, NSLICE)])
        plsc.subcore_barrier()

        pltpu.async_copy(vals_hbm.at[pl.ds(ebase, IW)], vals0_v, dsem0)

        @pl.loop(0, TROWS // 2)
        def _(g):
            q0 = 2 * g
            pltpu.async_copy(vals_hbm.at[pl.ds(ebase + (q0 + 1) * IW, IW)],
                             vals1_v, dsem1)
            pltpu.make_async_copy(vals_hbm.at[pl.ds(ebase, IW)], vals0_v,
                                  dsem0).wait()
            pltpu.sync_copy(vals0_v, acc.at[idx_v.at[q0]], add=True)

            @pl.when(g < TROWS // 2 - 1)
            def _():
                pltpu.async_copy(
                    vals_hbm.at[pl.ds(ebase + (q0 + 2) * IW, IW)],
                    vals0_v, dsem0)
            pltpu.make_async_copy(vals_hbm.at[pl.ds(ebase, IW)], vals1_v,
                                  dsem1).wait()
            pltpu.sync_copy(vals1_v, acc.at[idx_v.at[q0 + 1]], add=True)

        plsc.subcore_barrier()
        pltpu.sync_copy(acc.at[pl.ds(sid * NSLICE, NSLICE)],
                        out_hbm.at[p, cid, pl.ds(sid * NSLICE, NSLICE)])
        plsc.subcore_barrier()


def _scatter_sc(s, t0, t1, t2, j):
    mesh = plsc.VectorSubcoreMesh(core_axis_name="c", subcore_axis_name="s")
    j2 = j.reshape(JROWS, IW)
    zeros = jnp.zeros((N_ACC, ND), jnp.float32)
    run = pl.kernel(
        _scatter_body,
        out_type=jax.ShapeDtypeStruct((4, NC, N_ACC, ND), jnp.float32),
        mesh=mesh,
        scratch_types=[
            pltpu.VMEM((TROWS, IW), jnp.int32),
            pltpu.VMEM((IW, ND), jnp.float32),
            pltpu.VMEM((IW, ND), jnp.float32),
            pltpu.VMEM_SHARED((N_ACC, ND), jnp.float32),
            pltpu.SemaphoreType.DMA,
            pltpu.SemaphoreType.DMA,
        ],
    )
    return run(s, t0, t1, t2, j2, zeros)


def _final_kernel(f_ref, p_ref, wh1_ref, bh1_ref, wh2_ref, bh2_ref,
                  h0_ref, v0_ref, v1_ref, v2_ref):
    f32 = jnp.float32
    agg = p_ref[0, 0] + p_ref[0, 1]
    hh = _silu(
        jnp.dot(f_ref[...], wh1_ref[0:ND, :], preferred_element_type=f32)
        + jnp.dot(agg, wh1_ref[ND:, :], preferred_element_type=f32)
        + bh1_ref[...]
    )
    h0_ref[...] = jnp.dot(hh, wh2_ref[...], preferred_element_type=f32) \
        + bh2_ref[...]
    v0_ref[...] = p_ref[1, 0] + p_ref[1, 1]
    v1_ref[...] = p_ref[2, 0] + p_ref[2, 1]
    v2_ref[...] = p_ref[3, 0] + p_ref[3, 1]


def _final_tc(f, partials, Wh1, bh1, Wh2, bh2):
    nb = lambda: pl.BlockSpec((BF, ND), lambda b: (b, 0))
    full = lambda *shape: pl.BlockSpec(shape, lambda b: (0,) * len(shape))
    fs = jax.ShapeDtypeStruct((N, ND), jnp.float32)
    return pl.pallas_call(
        _final_kernel,
        grid=(NF_BLOCKS,),
        in_specs=[
            nb(),
            pl.BlockSpec((4, NC, BF, ND), lambda b: (0, 0, b, 0)),
            full(2 * ND, ED), full(1, ED), full(ED, ND), full(1, ND),
        ],
        out_specs=[nb()] * 4,
        out_shape=[fs] * 4,
    )(f, partials, Wh1, bh1.reshape(1, ED), Wh2, bh2.reshape(1, ND))


def kernel(species, edge_index, edge_attr, edge_vec,
           Wa, ba, Wb1, bb1, Wb2, bb2,
           Ws1, bs1, Ws2, bs2,
           Wh1, bh1, Wh2, bh2,
           Wv1, bv1, Wv2, bv2):
    npad = E_PAD - E
    pad_idx = (jnp.arange(npad, dtype=jnp.int32) * 8) % N
    i = jnp.concatenate([edge_index[0], pad_idx])
    j = jnp.concatenate([edge_index[1], pad_idx])
    edge_attr_p = jnp.pad(edge_attr, ((0, npad), (0, 0)))
    edge_vec_p = jnp.pad(edge_vec, ((0, npad), (0, 0)))

    f = _node_embed(species, Wa, ba)
    fi, fj = _gather_sc(f, i, j)
    ea_p, s, t0, t1, t2 = _edge_mlps(
        fi, fj, edge_attr_p, edge_vec_p,
        Wb1, bb1, Wb2, bb2, Ws1, bs1, Ws2, bs2, Wv1, bv1, Wv2, bv2)
    partials = _scatter_sc(s, t0, t1, t2, j)
    h0, v00, v01, v02 = _final_tc(f, partials, Wh1, bh1, Wh2, bh2)
    v0 = jnp.stack([v00, v01, v02], axis=-1)
    return (h0, v0, ea_p[:E])


# speedup vs baseline: 25.6798x; 1.3420x over previous
import functools

import jax
import jax.numpy as jnp
from jax import lax
from jax.experimental import pallas as pl
from jax.experimental.pallas import tpu as pltpu
from jax.experimental.pallas import tpu_sc as plsc

N = 10000
E = 320000
ND = 128
IED = 16
ED = 128

NC = 2
NS = 16
NW = NC * NS

N_ACC = 10240

GCHUNK = 320
PER_TILE = 10240
GITERS = PER_TILE // GCHUNK
GITERS_LAST = (E - 31 * PER_TILE) // GCHUNK

IW = 128
JROWS = E // IW
JROWS_PAD = NW * (PER_TILE // IW)
TROWS = PER_TILE // IW
TROWS_LAST = JROWS - 31 * TROWS
NSLICE = N_ACC // NS

BD = 1280
ND_BLOCKS = E // BD

BF = 1000
NF_BLOCKS = N // BF


def _silu(x):
    return x * jax.nn.sigmoid(x)


def _node_embed_kernel(species_ref, wa_ref, ba_ref, f_ref):
    f_ref[...] = (
        jnp.dot(species_ref[...], wa_ref[...],
                preferred_element_type=jnp.float32)
        + ba_ref[...]
    )


def _node_embed(species, Wa, ba):
    return pl.pallas_call(
        _node_embed_kernel,
        out_shape=jax.ShapeDtypeStruct((N, ND), jnp.float32),
    )(species, Wa, ba.reshape(1, ND))


def _gather_body(f_hbm, i_hbm, j_hbm, fi_hbm, fj_hbm,
                 idxi_v, idxj_v, rowsi_v, rowsj_v,
                 gsem, wsem_i, wsem_j):
    cid = lax.axis_index("c")
    sid = lax.axis_index("s")
    wid = cid * NS + sid
    base = wid * PER_TILE
    niter = jnp.where(wid == NW - 1, GITERS_LAST, GITERS)

    @pl.loop(0, niter)
    def _(k):
        off = base + k * GCHUNK
        pltpu.sync_copy(i_hbm.at[pl.ds(off, GCHUNK)], idxi_v)
        pltpu.sync_copy(j_hbm.at[pl.ds(off, GCHUNK)], idxj_v)

        @pl.when(k > 0)
        def _():
            pltpu.make_async_copy(rowsi_v, fi_hbm.at[pl.ds(off, GCHUNK)],
                                  wsem_i).wait()
        pltpu.async_copy(f_hbm.at[idxi_v], rowsi_v, gsem).wait()
        pltpu.async_copy(rowsi_v, fi_hbm.at[pl.ds(off, GCHUNK)], wsem_i)

        @pl.when(k > 0)
        def _():
            pltpu.make_async_copy(rowsj_v, fj_hbm.at[pl.ds(off, GCHUNK)],
                                  wsem_j).wait()
        pltpu.async_copy(f_hbm.at[idxj_v], rowsj_v, gsem).wait()
        pltpu.async_copy(rowsj_v, fj_hbm.at[pl.ds(off, GCHUNK)], wsem_j)

    pltpu.make_async_copy(rowsi_v, fi_hbm.at[pl.ds(base, GCHUNK)],
                          wsem_i).wait()
    pltpu.make_async_copy(rowsj_v, fj_hbm.at[pl.ds(base, GCHUNK)],
                          wsem_j).wait()


def _gather_sc(f, i, j):
    mesh = plsc.VectorSubcoreMesh(core_axis_name="c", subcore_axis_name="s")
    fs = jax.ShapeDtypeStruct((E, ND), jnp.float32)
    run = pl.kernel(
        _gather_body,
        out_type=(fs, fs),
        mesh=mesh,
        scratch_types=[
            pltpu.VMEM((GCHUNK,), jnp.int32),
            pltpu.VMEM((GCHUNK,), jnp.int32),
            pltpu.VMEM((GCHUNK, ND), jnp.float32),
            pltpu.VMEM((GCHUNK, ND), jnp.float32),
            pltpu.SemaphoreType.DMA,
            pltpu.SemaphoreType.DMA,
            pltpu.SemaphoreType.DMA,
        ],
    )
    return run(f, i, j)


def _edge_mlp_kernel(fi_ref, fj_ref, eattr_ref, evec_ref,
                     wb1_ref, bb1_ref, wb2_ref, bb2_ref,
                     w1_ref, b1_ref, w2_ref, b2_ref,
                     ea_ref, s_ref, t0_ref, t1_ref, t2_ref):
    f32 = jnp.float32
    bf16 = jnp.bfloat16
    fi32 = fi_ref[...]
    fi = fi32.astype(bf16)
    fj = fj_ref[...].astype(bf16)

    eh = _silu(jnp.dot(eattr_ref[...].astype(bf16), wb1_ref[...],
                       preferred_element_type=f32) + bb1_ref[...])
    ea = jnp.dot(eh.astype(bf16), wb2_ref[...],
                 preferred_element_type=f32) + bb2_ref[...]
    ea_ref[...] = ea

    h = _silu(
        jnp.dot(fi, w1_ref[0:ND, :], preferred_element_type=f32)
        + jnp.dot(fj, w1_ref[ND:2 * ND, :], preferred_element_type=f32)
        + jnp.dot(ea.astype(bf16), w1_ref[2 * ND:, :],
                  preferred_element_type=f32)
        + b1_ref[...]
    ).astype(bf16)
    sp = jnp.dot(h, w2_ref[...], preferred_element_type=f32) + b2_ref[...]

    s_ref[...] = sp[:, 0:ND] * fi32
    pv = sp[:, ND:]
    ev = evec_ref[...]
    t0_ref[...] = pv * ev[:, 0:1]
    t1_ref[...] = pv * ev[:, 1:2]
    t2_ref[...] = pv * ev[:, 2:3]


def _edge_mlps(fi, fj, edge_attr, edge_vec,
               Wb1, bb1, Wb2, bb2, Ws1, bs1, Ws2, bs2, Wv1, bv1, Wv2, bv2):
    bf16 = jnp.bfloat16
    w1 = jnp.concatenate([Ws1, Wv1], axis=1).astype(bf16)
    b1 = jnp.concatenate([bs1, bv1]).reshape(1, 2 * ND)
    w2 = jnp.zeros((2 * ND, 2 * ND), jnp.float32)
    w2 = w2.at[0:ND, 0:ND].set(Ws2).at[ND:, ND:].set(Wv2).astype(bf16)
    b2 = jnp.concatenate([bs2, bv2]).reshape(1, 2 * ND)

    eb = lambda w: pl.BlockSpec((BD, w), lambda b: (b, 0))
    full = lambda *shape: pl.BlockSpec(shape, lambda b: (0,) * len(shape))
    out_spec = pl.BlockSpec((BD, ND), lambda b: (b, 0))
    fs = jax.ShapeDtypeStruct((E, ND), jnp.float32)
    return pl.pallas_call(
        _edge_mlp_kernel,
        grid=(ND_BLOCKS,),
        in_specs=[
            eb(ND), eb(ND), eb(IED), eb(3),
            full(IED, ED), full(1, ED), full(ED, ED), full(1, ED),
            full(2 * ND + ED, 2 * ND), full(1, 2 * ND),
            full(2 * ND, 2 * ND), full(1, 2 * ND),
        ],
        out_specs=[out_spec] * 5,
        out_shape=[fs] * 5,
    )(fi, fj, edge_attr, edge_vec,
      Wb1.astype(bf16), bb1.reshape(1, ED), Wb2.astype(bf16),
      bb2.reshape(1, ED), w1, b1, w2, b2)


def _scatter_body(s_hbm, t0_hbm, t1_hbm, t2_hbm, j2_hbm, z_hbm, out_hbm,
                  idx_v, vals0_v, vals1_v, acc, dsem0, dsem1):
    cid = lax.axis_index("c")
    sid = lax.axis_index("s")
    tile = cid * NS + sid
    row0 = tile * TROWS
    ebase = tile * PER_TILE
    pairs = jnp.where(tile == NW - 1, TROWS_LAST // 2, TROWS // 2)

    pltpu.sync_copy(j2_hbm.at[pl.ds(row0, TROWS)], idx_v)

    for p, vals_hbm in enumerate((s_hbm, t0_hbm, t1_hbm, t2_hbm)):
        pltpu.sync_copy(z_hbm.at[pl.ds(sid * NSLICE, NSLICE)],
                        acc.at[pl.ds(sid * NSLICE, NSLICE)])
        plsc.subcore_barrier()

        pltpu.async_copy(vals_hbm.at[pl.ds(ebase, IW)], vals0_v, dsem0)

        @pl.loop(0, pairs)
        def _(g):
            q0 = 2 * g
            pltpu.async_copy(vals_hbm.at[pl.ds(ebase + (q0 + 1) * IW, IW)],
                             vals1_v, dsem1)
            pltpu.make_async_copy(vals_hbm.at[pl.ds(ebase, IW)], vals0_v,
                                  dsem0).wait()
            pltpu.sync_copy(vals0_v, acc.at[idx_v.at[q0]], add=True)

            @pl.when(g < pairs - 1)
            def _():
                pltpu.async_copy(
                    vals_hbm.at[pl.ds(ebase + (q0 + 2) * IW, IW)],
                    vals0_v, dsem0)
            pltpu.make_async_copy(vals_hbm.at[pl.ds(ebase, IW)], vals1_v,
                                  dsem1).wait()
            pltpu.sync_copy(vals1_v, acc.at[idx_v.at[q0 + 1]], add=True)

        plsc.subcore_barrier()
        pltpu.sync_copy(acc.at[pl.ds(sid * NSLICE, NSLICE)],
                        out_hbm.at[p, cid, pl.ds(sid * NSLICE, NSLICE)])
        plsc.subcore_barrier()


def _scatter_sc(s, t0, t1, t2, j):
    mesh = plsc.VectorSubcoreMesh(core_axis_name="c", subcore_axis_name="s")
    j2 = jnp.concatenate(
        [j, jnp.zeros(JROWS_PAD * IW - E, jnp.int32)]).reshape(JROWS_PAD, IW)
    zeros = jnp.zeros((N_ACC, ND), jnp.float32)
    run = pl.kernel(
        _scatter_body,
        out_type=jax.ShapeDtypeStruct((4, NC, N_ACC, ND), jnp.float32),
        mesh=mesh,
        scratch_types=[
            pltpu.VMEM((TROWS, IW), jnp.int32),
            pltpu.VMEM((IW, ND), jnp.float32),
            pltpu.VMEM((IW, ND), jnp.float32),
            pltpu.VMEM_SHARED((N_ACC, ND), jnp.float32),
            pltpu.SemaphoreType.DMA,
            pltpu.SemaphoreType.DMA,
        ],
    )
    return run(s, t0, t1, t2, j2, zeros)


def _final_kernel(f_ref, p_ref, wh1_ref, bh1_ref, wh2_ref, bh2_ref,
                  h0_ref, v0_ref, v1_ref, v2_ref):
    f32 = jnp.float32
    agg = p_ref[0, 0] + p_ref[0, 1]
    hh = _silu(
        jnp.dot(f_ref[...], wh1_ref[0:ND, :], preferred_element_type=f32)
        + jnp.dot(agg, wh1_ref[ND:, :], preferred_element_type=f32)
        + bh1_ref[...]
    )
    h0_ref[...] = jnp.dot(hh, wh2_ref[...], preferred_element_type=f32) \
        + bh2_ref[...]
    v0_ref[...] = p_ref[1, 0] + p_ref[1, 1]
    v1_ref[...] = p_ref[2, 0] + p_ref[2, 1]
    v2_ref[...] = p_ref[3, 0] + p_ref[3, 1]


def _final_tc(f, partials, Wh1, bh1, Wh2, bh2):
    nb = lambda: pl.BlockSpec((BF, ND), lambda b: (b, 0))
    full = lambda *shape: pl.BlockSpec(shape, lambda b: (0,) * len(shape))
    fs = jax.ShapeDtypeStruct((N, ND), jnp.float32)
    return pl.pallas_call(
        _final_kernel,
        grid=(NF_BLOCKS,),
        in_specs=[
            nb(),
            pl.BlockSpec((4, NC, BF, ND), lambda b: (0, 0, b, 0)),
            full(2 * ND, ED), full(1, ED), full(ED, ND), full(1, ND),
        ],
        out_specs=[nb()] * 4,
        out_shape=[fs] * 4,
    )(f, partials, Wh1, bh1.reshape(1, ED), Wh2, bh2.reshape(1, ND))


def kernel(species, edge_index, edge_attr, edge_vec,
           Wa, ba, Wb1, bb1, Wb2, bb2,
           Ws1, bs1, Ws2, bs2,
           Wh1, bh1, Wh2, bh2,
           Wv1, bv1, Wv2, bv2):
    i = edge_index[0]
    j = edge_index[1]

    f = _node_embed(species, Wa, ba)
    fi, fj = _gather_sc(f, i, j)
    ea, s, t0, t1, t2 = _edge_mlps(
        fi, fj, edge_attr, edge_vec,
        Wb1, bb1, Wb2, bb2, Ws1, bs1, Ws2, bs2, Wv1, bv1, Wv2, bv2)
    partials = _scatter_sc(s, t0, t1, t2, j)
    h0, v00, v01, v02 = _final_tc(f, partials, Wh1, bh1, Wh2, bh2)
    v0 = jnp.stack([v00, v01, v02], axis=-1)
    return (h0, v0, ea)


# speedup vs baseline: 26.4909x; 1.0316x over previous
import functools

import jax
import jax.numpy as jnp
from jax import lax
from jax.experimental import pallas as pl
from jax.experimental.pallas import tpu as pltpu
from jax.experimental.pallas import tpu_sc as plsc

N = 10000
E = 320000
ND = 128
IED = 16
ED = 128

NC = 2
NS = 16
NW = NC * NS

EH = E // 2

N_ACC = 10240

GCHUNK = 320
PER_TILE = 5120
GITERS = PER_TILE // GCHUNK
GITERS_LAST = (EH - 31 * PER_TILE) // GCHUNK

IW = 128
JROWS = EH // IW
JROWS_PAD = NW * (PER_TILE // IW)
TROWS = PER_TILE // IW
TROWS_LAST = JROWS - 31 * TROWS
NSLICE = N_ACC // NS

BD = 2000
ND_BLOCKS = EH // BD

BF = 1000
NF_BLOCKS = N // BF


def _silu(x):
    return x * jax.nn.sigmoid(x)


def _node_embed_kernel(species_ref, wa_ref, ba_ref, f_ref):
    f_ref[...] = (
        jnp.dot(species_ref[...], wa_ref[...],
                preferred_element_type=jnp.float32)
        + ba_ref[...]
    )


def _node_embed(species, Wa, ba):
    return pl.pallas_call(
        _node_embed_kernel,
        out_shape=jax.ShapeDtypeStruct((N, ND), jnp.float32),
    )(species, Wa, ba.reshape(1, ND))


def _gather_body(f_hbm, i_hbm, j_hbm, fi_hbm, fj_hbm,
                 idxi_v, idxj_v, rowsi_v, rowsj_v,
                 gsem, wsem_i, wsem_j):
    cid = lax.axis_index("c")
    sid = lax.axis_index("s")
    wid = cid * NS + sid
    base = wid * PER_TILE
    niter = jnp.where(wid == NW - 1, GITERS_LAST, GITERS)

    @pl.loop(0, niter)
    def _(k):
        off = base + k * GCHUNK
        pltpu.sync_copy(i_hbm.at[pl.ds(off, GCHUNK)], idxi_v)
        pltpu.sync_copy(j_hbm.at[pl.ds(off, GCHUNK)], idxj_v)

        @pl.when(k > 0)
        def _():
            pltpu.make_async_copy(rowsi_v, fi_hbm.at[pl.ds(off, GCHUNK)],
                                  wsem_i).wait()
        pltpu.async_copy(f_hbm.at[idxi_v], rowsi_v, gsem).wait()
        pltpu.async_copy(rowsi_v, fi_hbm.at[pl.ds(off, GCHUNK)], wsem_i)

        @pl.when(k > 0)
        def _():
            pltpu.make_async_copy(rowsj_v, fj_hbm.at[pl.ds(off, GCHUNK)],
                                  wsem_j).wait()
        pltpu.async_copy(f_hbm.at[idxj_v], rowsj_v, gsem).wait()
        pltpu.async_copy(rowsj_v, fj_hbm.at[pl.ds(off, GCHUNK)], wsem_j)

    pltpu.make_async_copy(rowsi_v, fi_hbm.at[pl.ds(base, GCHUNK)],
                          wsem_i).wait()
    pltpu.make_async_copy(rowsj_v, fj_hbm.at[pl.ds(base, GCHUNK)],
                          wsem_j).wait()


def _gather_sc(f, i, j):
    mesh = plsc.VectorSubcoreMesh(core_axis_name="c", subcore_axis_name="s")
    fs = jax.ShapeDtypeStruct((EH, ND), jnp.float32)
    run = pl.kernel(
        _gather_body,
        out_type=(fs, fs),
        mesh=mesh,
        scratch_types=[
            pltpu.VMEM((GCHUNK,), jnp.int32),
            pltpu.VMEM((GCHUNK,), jnp.int32),
            pltpu.VMEM((GCHUNK, ND), jnp.float32),
            pltpu.VMEM((GCHUNK, ND), jnp.float32),
            pltpu.SemaphoreType.DMA,
            pltpu.SemaphoreType.DMA,
            pltpu.SemaphoreType.DMA,
        ],
    )
    return run(f, i, j)


def _edge_mlp_kernel(fi_ref, fj_ref, eattr_ref, evec_ref,
                     wb1_ref, bb1_ref, wb2_ref, bb2_ref,
                     w1_ref, b1_ref, w2_ref, b2_ref,
                     ea_ref, s_ref, t0_ref, t1_ref, t2_ref):
    f32 = jnp.float32
    bf16 = jnp.bfloat16
    fi32 = fi_ref[...]
    fi = fi32.astype(bf16)
    fj = fj_ref[...].astype(bf16)

    eh = _silu(jnp.dot(eattr_ref[...].astype(bf16), wb1_ref[...],
                       preferred_element_type=f32) + bb1_ref[...])
    ea = jnp.dot(eh.astype(bf16), wb2_ref[...],
                 preferred_element_type=f32) + bb2_ref[...]
    ea_ref[...] = ea

    h = _silu(
        jnp.dot(fi, w1_ref[0:ND, :], preferred_element_type=f32)
        + jnp.dot(fj, w1_ref[ND:2 * ND, :], preferred_element_type=f32)
        + jnp.dot(ea.astype(bf16), w1_ref[2 * ND:, :],
                  preferred_element_type=f32)
        + b1_ref[...]
    ).astype(bf16)
    sp = jnp.dot(h, w2_ref[...], preferred_element_type=f32) + b2_ref[...]

    s_ref[...] = sp[:, 0:ND] * fi32
    pv = sp[:, ND:]
    ev = evec_ref[...]
    t0_ref[...] = pv * ev[:, 0:1]
    t1_ref[...] = pv * ev[:, 1:2]
    t2_ref[...] = pv * ev[:, 2:3]


def _edge_mlps(fi, fj, edge_attr, edge_vec,
               Wb1, bb1, Wb2, bb2, Ws1, bs1, Ws2, bs2, Wv1, bv1, Wv2, bv2):
    bf16 = jnp.bfloat16
    w1 = jnp.concatenate([Ws1, Wv1], axis=1).astype(bf16)
    b1 = jnp.concatenate([bs1, bv1]).reshape(1, 2 * ND)
    w2 = jnp.zeros((2 * ND, 2 * ND), jnp.float32)
    w2 = w2.at[0:ND, 0:ND].set(Ws2).at[ND:, ND:].set(Wv2).astype(bf16)
    b2 = jnp.concatenate([bs2, bv2]).reshape(1, 2 * ND)

    eb = lambda w: pl.BlockSpec((BD, w), lambda b: (b, 0))
    full = lambda *shape: pl.BlockSpec(shape, lambda b: (0,) * len(shape))
    out_spec = pl.BlockSpec((BD, ND), lambda b: (b, 0))
    fs = jax.ShapeDtypeStruct((EH, ND), jnp.float32)
    return pl.pallas_call(
        _edge_mlp_kernel,
        grid=(ND_BLOCKS,),
        in_specs=[
            eb(ND), eb(ND), eb(IED), eb(3),
            full(IED, ED), full(1, ED), full(ED, ED), full(1, ED),
            full(2 * ND + ED, 2 * ND), full(1, 2 * ND),
            full(2 * ND, 2 * ND), full(1, 2 * ND),
        ],
        out_specs=[out_spec] * 5,
        out_shape=[fs] * 5,
    )(fi, fj, edge_attr, edge_vec,
      Wb1.astype(bf16), bb1.reshape(1, ED), Wb2.astype(bf16),
      bb2.reshape(1, ED), w1, b1, w2, b2)


def _scatter_body(s_hbm, t0_hbm, t1_hbm, t2_hbm, j2_hbm, z_hbm, out_hbm,
                  idx_v, vals0_v, vals1_v, acc, dsem0, dsem1):
    cid = lax.axis_index("c")
    sid = lax.axis_index("s")
    tile = cid * NS + sid
    row0 = tile * TROWS
    ebase = tile * PER_TILE
    pairs = jnp.where(tile == NW - 1, TROWS_LAST // 2, TROWS // 2)

    pltpu.sync_copy(j2_hbm.at[pl.ds(row0, TROWS)], idx_v)

    for p, vals_hbm in enumerate((s_hbm, t0_hbm, t1_hbm, t2_hbm)):
        pltpu.sync_copy(z_hbm.at[pl.ds(sid * NSLICE, NSLICE)],
                        acc.at[pl.ds(sid * NSLICE, NSLICE)])
        plsc.subcore_barrier()

        pltpu.async_copy(vals_hbm.at[pl.ds(ebase, IW)], vals0_v, dsem0)

        @pl.loop(0, pairs)
        def _(g):
            q0 = 2 * g
            pltpu.async_copy(vals_hbm.at[pl.ds(ebase + (q0 + 1) * IW, IW)],
                             vals1_v, dsem1)
            pltpu.make_async_copy(vals_hbm.at[pl.ds(ebase, IW)], vals0_v,
                                  dsem0).wait()
            pltpu.sync_copy(vals0_v, acc.at[idx_v.at[q0]], add=True)

            @pl.when(g < pairs - 1)
            def _():
                pltpu.async_copy(
                    vals_hbm.at[pl.ds(ebase + (q0 + 2) * IW, IW)],
                    vals0_v, dsem0)
            pltpu.make_async_copy(vals_hbm.at[pl.ds(ebase, IW)], vals1_v,
                                  dsem1).wait()
            pltpu.sync_copy(vals1_v, acc.at[idx_v.at[q0 + 1]], add=True)

        plsc.subcore_barrier()
        pltpu.sync_copy(acc.at[pl.ds(sid * NSLICE, NSLICE)],
                        out_hbm.at[p, cid, pl.ds(sid * NSLICE, NSLICE)])
        plsc.subcore_barrier()


def _scatter_sc(s, t0, t1, t2, j):
    mesh = plsc.VectorSubcoreMesh(core_axis_name="c", subcore_axis_name="s")
    j2 = jnp.concatenate(
        [j, jnp.zeros(JROWS_PAD * IW - EH, jnp.int32)]).reshape(JROWS_PAD, IW)
    zeros = jnp.zeros((N_ACC, ND), jnp.float32)
    run = pl.kernel(
        _scatter_body,
        out_type=jax.ShapeDtypeStruct((4, NC, N_ACC, ND), jnp.float32),
        mesh=mesh,
        scratch_types=[
            pltpu.VMEM((TROWS, IW), jnp.int32),
            pltpu.VMEM((IW, ND), jnp.float32),
            pltpu.VMEM((IW, ND), jnp.float32),
            pltpu.VMEM_SHARED((N_ACC, ND), jnp.float32),
            pltpu.SemaphoreType.DMA,
            pltpu.SemaphoreType.DMA,
        ],
    )
    return run(s, t0, t1, t2, j2, zeros)


def _final_kernel(f_ref, p0_ref, p1_ref, wh1_ref, bh1_ref, wh2_ref, bh2_ref,
                  h0_ref, v0_ref, v1_ref, v2_ref):
    f32 = jnp.float32
    agg = (p0_ref[0, 0] + p0_ref[0, 1]) + (p1_ref[0, 0] + p1_ref[0, 1])
    hh = _silu(
        jnp.dot(f_ref[...], wh1_ref[0:ND, :], preferred_element_type=f32)
        + jnp.dot(agg, wh1_ref[ND:, :], preferred_element_type=f32)
        + bh1_ref[...]
    )
    h0_ref[...] = jnp.dot(hh, wh2_ref[...], preferred_element_type=f32) \
        + bh2_ref[...]
    v0_ref[...] = (p0_ref[1, 0] + p0_ref[1, 1]) + (p1_ref[1, 0] + p1_ref[1, 1])
    v1_ref[...] = (p0_ref[2, 0] + p0_ref[2, 1]) + (p1_ref[2, 0] + p1_ref[2, 1])
    v2_ref[...] = (p0_ref[3, 0] + p0_ref[3, 1]) + (p1_ref[3, 0] + p1_ref[3, 1])


def _final_tc(f, partials0, partials1, Wh1, bh1, Wh2, bh2):
    nb = lambda: pl.BlockSpec((BF, ND), lambda b: (b, 0))
    full = lambda *shape: pl.BlockSpec(shape, lambda b: (0,) * len(shape))
    ps = pl.BlockSpec((4, NC, BF, ND), lambda b: (0, 0, b, 0))
    fs = jax.ShapeDtypeStruct((N, ND), jnp.float32)
    return pl.pallas_call(
        _final_kernel,
        grid=(NF_BLOCKS,),
        in_specs=[
            nb(), ps, ps,
            full(2 * ND, ED), full(1, ED), full(ED, ND), full(1, ND),
        ],
        out_specs=[nb()] * 4,
        out_shape=[fs] * 4,
    )(f, partials0, partials1,
      Wh1, bh1.reshape(1, ED), Wh2, bh2.reshape(1, ND))


def kernel(species, edge_index, edge_attr, edge_vec,
           Wa, ba, Wb1, bb1, Wb2, bb2,
           Ws1, bs1, Ws2, bs2,
           Wh1, bh1, Wh2, bh2,
           Wv1, bv1, Wv2, bv2):
    i = edge_index[0]
    j = edge_index[1]

    f = _node_embed(species, Wa, ba)

    ea_h, part_h = [], []
    for h in range(2):
        sl = slice(h * EH, (h + 1) * EH)
        fi, fj = _gather_sc(f, i[sl], j[sl])
        ea, s, t0, t1, t2 = _edge_mlps(
            fi, fj, edge_attr[sl], edge_vec[sl],
            Wb1, bb1, Wb2, bb2, Ws1, bs1, Ws2, bs2, Wv1, bv1, Wv2, bv2)
        ea_h.append(ea)
        part_h.append(_scatter_sc(s, t0, t1, t2, j[sl]))

    h0, v00, v01, v02 = _final_tc(f, part_h[0], part_h[1],
                                  Wh1, bh1, Wh2, bh2)
    v0 = jnp.stack([v00, v01, v02], axis=-1)
    return (h0, v0, jnp.concatenate(ea_h, axis=0))


# speedup vs baseline: 30.1646x; 1.1387x over previous
import functools

import jax
import jax.numpy as jnp
from jax import lax
from jax.experimental import pallas as pl
from jax.experimental.pallas import tpu as pltpu
from jax.experimental.pallas import tpu_sc as plsc

N = 10000
E = 320000
ND = 128
IED = 16
ED = 128

NC = 2
NS = 16
NW = NC * NS

EH = E // 2

N_ACC = 10240

GCHUNK = 320
PER_TILE = 5120
GITERS = PER_TILE // GCHUNK
GITERS_LAST = (EH - 31 * PER_TILE) // GCHUNK

IW = 128
JROWS = EH // IW
JROWS_PAD = NW * (PER_TILE // IW)
TROWS = PER_TILE // IW
TROWS_LAST = JROWS - 31 * TROWS
NSLICE = N_ACC // NS

BD = 1280
ND_BLOCKS = EH // BD

BF = 1000
NF_BLOCKS = N // BF


def _silu(x):
    return x * jax.nn.sigmoid(x)


def _node_embed_kernel(species_ref, wa_ref, ba_ref, f_ref):
    f_ref[...] = (
        jnp.dot(species_ref[...], wa_ref[...],
                preferred_element_type=jnp.float32)
        + ba_ref[...]
    )


def _node_embed(species, Wa, ba):
    return pl.pallas_call(
        _node_embed_kernel,
        out_shape=jax.ShapeDtypeStruct((N, ND), jnp.float32),
    )(species, Wa, ba.reshape(1, ND))


def _gather_body(f_hbm, i_hbm, j_hbm, fi_hbm, fj_hbm,
                 idxi_v, idxj_v, rowsi_v, rowsj_v,
                 gsem, wsem_i, wsem_j):
    cid = lax.axis_index("c")
    sid = lax.axis_index("s")
    wid = cid * NS + sid
    base = wid * PER_TILE
    niter = jnp.where(wid == NW - 1, GITERS_LAST, GITERS)

    @pl.loop(0, niter)
    def _(k):
        off = base + k * GCHUNK
        pltpu.sync_copy(i_hbm.at[pl.ds(off, GCHUNK)], idxi_v)
        pltpu.sync_copy(j_hbm.at[pl.ds(off, GCHUNK)], idxj_v)

        @pl.when(k > 0)
        def _():
            pltpu.make_async_copy(rowsi_v, fi_hbm.at[pl.ds(off, GCHUNK)],
                                  wsem_i).wait()
        pltpu.async_copy(f_hbm.at[idxi_v], rowsi_v, gsem).wait()
        pltpu.async_copy(rowsi_v, fi_hbm.at[pl.ds(off, GCHUNK)], wsem_i)

        @pl.when(k > 0)
        def _():
            pltpu.make_async_copy(rowsj_v, fj_hbm.at[pl.ds(off, GCHUNK)],
                                  wsem_j).wait()
        pltpu.async_copy(f_hbm.at[idxj_v], rowsj_v, gsem).wait()
        pltpu.async_copy(rowsj_v, fj_hbm.at[pl.ds(off, GCHUNK)], wsem_j)

    pltpu.make_async_copy(rowsi_v, fi_hbm.at[pl.ds(base, GCHUNK)],
                          wsem_i).wait()
    pltpu.make_async_copy(rowsj_v, fj_hbm.at[pl.ds(base, GCHUNK)],
                          wsem_j).wait()


def _gather_sc(f, i, j):
    mesh = plsc.VectorSubcoreMesh(core_axis_name="c", subcore_axis_name="s")
    fs = jax.ShapeDtypeStruct((EH, ND), jnp.float32)
    run = pl.kernel(
        _gather_body,
        out_type=(fs, fs),
        mesh=mesh,
        scratch_types=[
            pltpu.VMEM((GCHUNK,), jnp.int32),
            pltpu.VMEM((GCHUNK,), jnp.int32),
            pltpu.VMEM((GCHUNK, ND), jnp.float32),
            pltpu.VMEM((GCHUNK, ND), jnp.float32),
            pltpu.SemaphoreType.DMA,
            pltpu.SemaphoreType.DMA,
            pltpu.SemaphoreType.DMA,
        ],
    )
    return run(f, i, j)


def _edge_mlp_kernel(fi_ref, fj_ref, eattr_ref, evec_ref,
                     wb1_ref, bb1_ref, wb2_ref, bb2_ref,
                     w1_ref, b1_ref, w2_ref, b2_ref,
                     ea_ref, s_ref, t0_ref, t1_ref, t2_ref):
    f32 = jnp.float32
    bf16 = jnp.bfloat16
    fi32 = fi_ref[...]
    fi = fi32.astype(bf16)
    fj = fj_ref[...].astype(bf16)

    eh = _silu(lax.dot_general(eattr_ref[...].astype(bf16), wb1_ref[...],
                               (((0,), (0,)), ((), ())),
                               preferred_element_type=f32) + bb1_ref[...])
    ea = jnp.dot(eh.astype(bf16), wb2_ref[...],
                 preferred_element_type=f32) + bb2_ref[...]
    ea_ref[...] = ea

    h = _silu(
        jnp.dot(fi, w1_ref[0:ND, :], preferred_element_type=f32)
        + jnp.dot(fj, w1_ref[ND:2 * ND, :], preferred_element_type=f32)
        + jnp.dot(ea.astype(bf16), w1_ref[2 * ND:, :],
                  preferred_element_type=f32)
        + b1_ref[...]
    ).astype(bf16)
    sp = jnp.dot(h, w2_ref[...], preferred_element_type=f32) + b2_ref[...]

    s_ref[...] = sp[:, 0:ND] * fi32
    pv = sp[:, ND:]
    ev = jnp.transpose(evec_ref[...])
    t0_ref[...] = pv * ev[:, 0:1]
    t1_ref[...] = pv * ev[:, 1:2]
    t2_ref[...] = pv * ev[:, 2:3]


def _edge_mlps(fi, fj, edge_attr, edge_vec,
               Wb1, bb1, Wb2, bb2, Ws1, bs1, Ws2, bs2, Wv1, bv1, Wv2, bv2):
    bf16 = jnp.bfloat16
    w1 = jnp.concatenate([Ws1, Wv1], axis=1).astype(bf16)
    b1 = jnp.concatenate([bs1, bv1]).reshape(1, 2 * ND)
    w2 = jnp.zeros((2 * ND, 2 * ND), jnp.float32)
    w2 = w2.at[0:ND, 0:ND].set(Ws2).at[ND:, ND:].set(Wv2).astype(bf16)
    b2 = jnp.concatenate([bs2, bv2]).reshape(1, 2 * ND)

    eb = lambda w: pl.BlockSpec((BD, w), lambda b: (b, 0))
    full = lambda *shape: pl.BlockSpec(shape, lambda b: (0,) * len(shape))
    out_spec = pl.BlockSpec((BD, ND), lambda b: (b, 0))
    fs = jax.ShapeDtypeStruct((EH, ND), jnp.float32)
    return pl.pallas_call(
        _edge_mlp_kernel,
        grid=(ND_BLOCKS,),
        in_specs=[
            eb(ND), eb(ND),
            pl.BlockSpec((IED, BD), lambda b: (0, b)),
            pl.BlockSpec((3, BD), lambda b: (0, b)),
            full(IED, ED), full(1, ED), full(ED, ED), full(1, ED),
            full(2 * ND + ED, 2 * ND), full(1, 2 * ND),
            full(2 * ND, 2 * ND), full(1, 2 * ND),
        ],
        out_specs=[out_spec] * 5,
        out_shape=[fs] * 5,
    )(fi, fj, edge_attr, edge_vec,
      Wb1.astype(bf16), bb1.reshape(1, ED), Wb2.astype(bf16),
      bb2.reshape(1, ED), w1, b1, w2, b2)


def _scatter_body(s_hbm, t0_hbm, t1_hbm, t2_hbm, j2_hbm, z_hbm, out_hbm,
                  idx_v, vals0_v, vals1_v, acc, dsem0, dsem1):
    cid = lax.axis_index("c")
    sid = lax.axis_index("s")
    tile = cid * NS + sid
    row0 = tile * TROWS
    ebase = tile * PER_TILE
    pairs = jnp.where(tile == NW - 1, TROWS_LAST // 2, TROWS // 2)

    pltpu.sync_copy(j2_hbm.at[pl.ds(row0, TROWS)], idx_v)

    for p, vals_hbm in enumerate((s_hbm, t0_hbm, t1_hbm, t2_hbm)):
        pltpu.sync_copy(z_hbm.at[pl.ds(sid * NSLICE, NSLICE)],
                        acc.at[pl.ds(sid * NSLICE, NSLICE)])
        plsc.subcore_barrier()

        pltpu.async_copy(vals_hbm.at[pl.ds(ebase, IW)], vals0_v, dsem0)

        @pl.loop(0, pairs)
        def _(g):
            q0 = 2 * g
            pltpu.async_copy(vals_hbm.at[pl.ds(ebase + (q0 + 1) * IW, IW)],
                             vals1_v, dsem1)
            pltpu.make_async_copy(vals_hbm.at[pl.ds(ebase, IW)], vals0_v,
                                  dsem0).wait()
            pltpu.sync_copy(vals0_v, acc.at[idx_v.at[q0]], add=True)

            @pl.when(g < pairs - 1)
            def _():
                pltpu.async_copy(
                    vals_hbm.at[pl.ds(ebase + (q0 + 2) * IW, IW)],
                    vals0_v, dsem0)
            pltpu.make_async_copy(vals_hbm.at[pl.ds(ebase, IW)], vals1_v,
                                  dsem1).wait()
            pltpu.sync_copy(vals1_v, acc.at[idx_v.at[q0 + 1]], add=True)

        plsc.subcore_barrier()
        pltpu.sync_copy(acc.at[pl.ds(sid * NSLICE, NSLICE)],
                        out_hbm.at[p, cid, pl.ds(sid * NSLICE, NSLICE)])
        plsc.subcore_barrier()


def _scatter_sc(s, t0, t1, t2, j):
    mesh = plsc.VectorSubcoreMesh(core_axis_name="c", subcore_axis_name="s")
    j2 = jnp.concatenate(
        [j, jnp.zeros(JROWS_PAD * IW - EH, jnp.int32)]).reshape(JROWS_PAD, IW)
    zeros = jnp.zeros((N_ACC, ND), jnp.float32)
    run = pl.kernel(
        _scatter_body,
        out_type=jax.ShapeDtypeStruct((4, NC, N_ACC, ND), jnp.float32),
        mesh=mesh,
        scratch_types=[
            pltpu.VMEM((TROWS, IW), jnp.int32),
            pltpu.VMEM((IW, ND), jnp.float32),
            pltpu.VMEM((IW, ND), jnp.float32),
            pltpu.VMEM_SHARED((N_ACC, ND), jnp.float32),
            pltpu.SemaphoreType.DMA,
            pltpu.SemaphoreType.DMA,
        ],
    )
    return run(s, t0, t1, t2, j2, zeros)


def _final_kernel(f_ref, p0_ref, p1_ref, wh1_ref, bh1_ref, wh2_ref, bh2_ref,
                  h0_ref, v0_ref, v1_ref, v2_ref):
    f32 = jnp.float32
    agg = (p0_ref[0, 0] + p0_ref[0, 1]) + (p1_ref[0, 0] + p1_ref[0, 1])
    hh = _silu(
        jnp.dot(f_ref[...], wh1_ref[0:ND, :], preferred_element_type=f32)
        + jnp.dot(agg, wh1_ref[ND:, :], preferred_element_type=f32)
        + bh1_ref[...]
    )
    h0_ref[...] = jnp.dot(hh, wh2_ref[...], preferred_element_type=f32) \
        + bh2_ref[...]
    v0_ref[...] = (p0_ref[1, 0] + p0_ref[1, 1]) + (p1_ref[1, 0] + p1_ref[1, 1])
    v1_ref[...] = (p0_ref[2, 0] + p0_ref[2, 1]) + (p1_ref[2, 0] + p1_ref[2, 1])
    v2_ref[...] = (p0_ref[3, 0] + p0_ref[3, 1]) + (p1_ref[3, 0] + p1_ref[3, 1])


def _final_tc(f, partials0, partials1, Wh1, bh1, Wh2, bh2):
    nb = lambda: pl.BlockSpec((BF, ND), lambda b: (b, 0))
    full = lambda *shape: pl.BlockSpec(shape, lambda b: (0,) * len(shape))
    ps = pl.BlockSpec((4, NC, BF, ND), lambda b: (0, 0, b, 0))
    fs = jax.ShapeDtypeStruct((N, ND), jnp.float32)
    return pl.pallas_call(
        _final_kernel,
        grid=(NF_BLOCKS,),
        in_specs=[
            nb(), ps, ps,
            full(2 * ND, ED), full(1, ED), full(ED, ND), full(1, ND),
        ],
        out_specs=[nb()] * 4,
        out_shape=[fs] * 4,
    )(f, partials0, partials1,
      Wh1, bh1.reshape(1, ED), Wh2, bh2.reshape(1, ND))


def kernel(species, edge_index, edge_attr, edge_vec,
           Wa, ba, Wb1, bb1, Wb2, bb2,
           Ws1, bs1, Ws2, bs2,
           Wh1, bh1, Wh2, bh2,
           Wv1, bv1, Wv2, bv2):
    i = edge_index[0]
    j = edge_index[1]

    f = _node_embed(species, Wa, ba)

    ea_h, part_h = [], []
    for h in range(2):
        sl = slice(h * EH, (h + 1) * EH)
        fi, fj = _gather_sc(f, i[sl], j[sl])
        ea, s, t0, t1, t2 = _edge_mlps(
            fi, fj, edge_attr[sl].T, edge_vec[sl].T,
            Wb1, bb1, Wb2, bb2, Ws1, bs1, Ws2, bs2, Wv1, bv1, Wv2, bv2)
        ea_h.append(ea)
        part_h.append(_scatter_sc(s, t0, t1, t2, j[sl]))

    h0, v00, v01, v02 = _final_tc(f, part_h[0], part_h[1],
                                  Wh1, bh1, Wh2, bh2)
    v0 = jnp.stack([v00, v01, v02], axis=-1)
    return (h0, v0, jnp.concatenate(ea_h, axis=0))


# speedup vs baseline: 30.5005x; 1.0111x over previous
import functools

import jax
import jax.numpy as jnp
from jax import lax
from jax.experimental import pallas as pl
from jax.experimental.pallas import tpu as pltpu
from jax.experimental.pallas import tpu_sc as plsc

N = 10000
E = 320000
ND = 128
IED = 16
ED = 128

NC = 2
NS = 16
NW = NC * NS

EH = E // 2

N_ACC = 10240

GCHUNK = 320
PER_TILE = 5120
GITERS = PER_TILE // GCHUNK
GITERS_LAST = (EH - 31 * PER_TILE) // GCHUNK

IW = 128
JROWS = EH // IW
JROWS_PAD = NW * (PER_TILE // IW)
TROWS = PER_TILE // IW
TROWS_LAST = JROWS - 31 * TROWS
NSLICE = N_ACC // NS

BD = 3200
ND_BLOCKS = EH // BD

BF = 1000
NF_BLOCKS = N // BF


def _silu(x):
    return x * jax.nn.sigmoid(x)


def _node_embed_kernel(species_ref, wa_ref, ba_ref, f_ref):
    f_ref[...] = (
        jnp.dot(species_ref[...], wa_ref[...],
                preferred_element_type=jnp.float32)
        + ba_ref[...]
    )


def _node_embed(species, Wa, ba):
    return pl.pallas_call(
        _node_embed_kernel,
        out_shape=jax.ShapeDtypeStruct((N, ND), jnp.float32),
    )(species, Wa, ba.reshape(1, ND))


def _gather_body(f_hbm, i_hbm, j_hbm, fi_hbm, fj_hbm,
                 idxi_v, idxj_v, rowsi_v, rowsj_v,
                 gsem, wsem_i, wsem_j):
    cid = lax.axis_index("c")
    sid = lax.axis_index("s")
    wid = cid * NS + sid
    base = wid * PER_TILE
    niter = jnp.where(wid == NW - 1, GITERS_LAST, GITERS)

    @pl.loop(0, niter)
    def _(k):
        off = base + k * GCHUNK
        pltpu.sync_copy(i_hbm.at[pl.ds(off, GCHUNK)], idxi_v)
        pltpu.sync_copy(j_hbm.at[pl.ds(off, GCHUNK)], idxj_v)

        @pl.when(k > 0)
        def _():
            pltpu.make_async_copy(rowsi_v, fi_hbm.at[pl.ds(off, GCHUNK)],
                                  wsem_i).wait()
        pltpu.async_copy(f_hbm.at[idxi_v], rowsi_v, gsem).wait()
        pltpu.async_copy(rowsi_v, fi_hbm.at[pl.ds(off, GCHUNK)], wsem_i)

        @pl.when(k > 0)
        def _():
            pltpu.make_async_copy(rowsj_v, fj_hbm.at[pl.ds(off, GCHUNK)],
                                  wsem_j).wait()
        pltpu.async_copy(f_hbm.at[idxj_v], rowsj_v, gsem).wait()
        pltpu.async_copy(rowsj_v, fj_hbm.at[pl.ds(off, GCHUNK)], wsem_j)

    pltpu.make_async_copy(rowsi_v, fi_hbm.at[pl.ds(base, GCHUNK)],
                          wsem_i).wait()
    pltpu.make_async_copy(rowsj_v, fj_hbm.at[pl.ds(base, GCHUNK)],
                          wsem_j).wait()


def _gather_sc(f, i, j):
    mesh = plsc.VectorSubcoreMesh(core_axis_name="c", subcore_axis_name="s")
    fs = jax.ShapeDtypeStruct((EH, ND), jnp.float32)
    run = pl.kernel(
        _gather_body,
        out_type=(fs, fs),
        mesh=mesh,
        scratch_types=[
            pltpu.VMEM((GCHUNK,), jnp.int32),
            pltpu.VMEM((GCHUNK,), jnp.int32),
            pltpu.VMEM((GCHUNK, ND), jnp.float32),
            pltpu.VMEM((GCHUNK, ND), jnp.float32),
            pltpu.SemaphoreType.DMA,
            pltpu.SemaphoreType.DMA,
            pltpu.SemaphoreType.DMA,
        ],
    )
    return run(f, i, j)


def _edge_mlp_kernel(fi_ref, fj_ref, eattr_ref, evec_ref,
                     wb1_ref, bb1_ref, wb2_ref, bb2_ref,
                     w1_ref, b1_ref, w2_ref, b2_ref,
                     ea_ref, s_ref, t0_ref, t1_ref, t2_ref):
    f32 = jnp.float32
    bf16 = jnp.bfloat16
    fi32 = fi_ref[...]
    fi = fi32.astype(bf16)
    fj = fj_ref[...].astype(bf16)

    eh = _silu(lax.dot_general(eattr_ref[...].astype(bf16), wb1_ref[...],
                               (((0,), (0,)), ((), ())),
                               preferred_element_type=f32) + bb1_ref[...])
    ea = jnp.dot(eh.astype(bf16), wb2_ref[...],
                 preferred_element_type=f32) + bb2_ref[...]
    ea_ref[...] = ea

    h = _silu(
        jnp.dot(fi, w1_ref[0:ND, :], preferred_element_type=f32)
        + jnp.dot(fj, w1_ref[ND:2 * ND, :], preferred_element_type=f32)
        + jnp.dot(ea.astype(bf16), w1_ref[2 * ND:, :],
                  preferred_element_type=f32)
        + b1_ref[...]
    ).astype(bf16)
    sp = jnp.dot(h, w2_ref[...], preferred_element_type=f32) + b2_ref[...]

    s_ref[...] = sp[:, 0:ND] * fi32
    pv = sp[:, ND:]
    ev = jnp.transpose(evec_ref[...])
    t0_ref[...] = pv * ev[:, 0:1]
    t1_ref[...] = pv * ev[:, 1:2]
    t2_ref[...] = pv * ev[:, 2:3]


def _edge_mlps(fi, fj, edge_attr, edge_vec,
               Wb1, bb1, Wb2, bb2, Ws1, bs1, Ws2, bs2, Wv1, bv1, Wv2, bv2):
    bf16 = jnp.bfloat16
    w1 = jnp.concatenate([Ws1, Wv1], axis=1).astype(bf16)
    b1 = jnp.concatenate([bs1, bv1]).reshape(1, 2 * ND)
    w2 = jnp.zeros((2 * ND, 2 * ND), jnp.float32)
    w2 = w2.at[0:ND, 0:ND].set(Ws2).at[ND:, ND:].set(Wv2).astype(bf16)
    b2 = jnp.concatenate([bs2, bv2]).reshape(1, 2 * ND)

    eb = lambda w: pl.BlockSpec((BD, w), lambda b: (b, 0))
    full = lambda *shape: pl.BlockSpec(shape, lambda b: (0,) * len(shape))
    out_spec = pl.BlockSpec((BD, ND), lambda b: (b, 0))
    fs = jax.ShapeDtypeStruct((EH, ND), jnp.float32)
    return pl.pallas_call(
        _edge_mlp_kernel,
        grid=(ND_BLOCKS,),
        in_specs=[
            eb(ND), eb(ND),
            pl.BlockSpec((IED, BD), lambda b: (0, b)),
            pl.BlockSpec((3, BD), lambda b: (0, b)),
            full(IED, ED), full(1, ED), full(ED, ED), full(1, ED),
            full(2 * ND + ED, 2 * ND), full(1, 2 * ND),
            full(2 * ND, 2 * ND), full(1, 2 * ND),
        ],
        out_specs=[out_spec] * 5,
        out_shape=[fs] * 5,
    )(fi, fj, edge_attr, edge_vec,
      Wb1.astype(bf16), bb1.reshape(1, ED), Wb2.astype(bf16),
      bb2.reshape(1, ED), w1, b1, w2, b2)


def _scatter_body(s_hbm, t0_hbm, t1_hbm, t2_hbm, j2_hbm, z_hbm, out_hbm,
                  idx_v, vals0_v, vals1_v, acc, dsem0, dsem1):
    cid = lax.axis_index("c")
    sid = lax.axis_index("s")
    tile = cid * NS + sid
    row0 = tile * TROWS
    ebase = tile * PER_TILE
    pairs = jnp.where(tile == NW - 1, TROWS_LAST // 2, TROWS // 2)

    pltpu.sync_copy(j2_hbm.at[pl.ds(row0, TROWS)], idx_v)

    for p, vals_hbm in enumerate((s_hbm, t0_hbm, t1_hbm, t2_hbm)):
        pltpu.sync_copy(z_hbm.at[pl.ds(sid * NSLICE, NSLICE)],
                        acc.at[pl.ds(sid * NSLICE, NSLICE)])
        plsc.subcore_barrier()

        pltpu.async_copy(vals_hbm.at[pl.ds(ebase, IW)], vals0_v, dsem0)

        @pl.loop(0, pairs)
        def _(g):
            q0 = 2 * g
            pltpu.async_copy(vals_hbm.at[pl.ds(ebase + (q0 + 1) * IW, IW)],
                             vals1_v, dsem1)
            pltpu.make_async_copy(vals_hbm.at[pl.ds(ebase, IW)], vals0_v,
                                  dsem0).wait()
            pltpu.sync_copy(vals0_v, acc.at[idx_v.at[q0]], add=True)

            @pl.when(g < pairs - 1)
            def _():
                pltpu.async_copy(
                    vals_hbm.at[pl.ds(ebase + (q0 + 2) * IW, IW)],
                    vals0_v, dsem0)
            pltpu.make_async_copy(vals_hbm.at[pl.ds(ebase, IW)], vals1_v,
                                  dsem1).wait()
            pltpu.sync_copy(vals1_v, acc.at[idx_v.at[q0 + 1]], add=True)

        plsc.subcore_barrier()
        pltpu.sync_copy(acc.at[pl.ds(sid * NSLICE, NSLICE)],
                        out_hbm.at[p, cid, pl.ds(sid * NSLICE, NSLICE)])
        plsc.subcore_barrier()


def _scatter_sc(s, t0, t1, t2, j):
    mesh = plsc.VectorSubcoreMesh(core_axis_name="c", subcore_axis_name="s")
    j2 = jnp.concatenate(
        [j, jnp.zeros(JROWS_PAD * IW - EH, jnp.int32)]).reshape(JROWS_PAD, IW)
    zeros = jnp.zeros((N_ACC, ND), jnp.float32)
    run = pl.kernel(
        _scatter_body,
        out_type=jax.ShapeDtypeStruct((4, NC, N_ACC, ND), jnp.float32),
        mesh=mesh,
        scratch_types=[
            pltpu.VMEM((TROWS, IW), jnp.int32),
            pltpu.VMEM((IW, ND), jnp.float32),
            pltpu.VMEM((IW, ND), jnp.float32),
            pltpu.VMEM_SHARED((N_ACC, ND), jnp.float32),
            pltpu.SemaphoreType.DMA,
            pltpu.SemaphoreType.DMA,
        ],
    )
    return run(s, t0, t1, t2, j2, zeros)


def _final_kernel(f_ref, p0_ref, p1_ref, wh1_ref, bh1_ref, wh2_ref, bh2_ref,
                  h0_ref, v0_ref, v1_ref, v2_ref):
    f32 = jnp.float32
    agg = (p0_ref[0, 0] + p0_ref[0, 1]) + (p1_ref[0, 0] + p1_ref[0, 1])
    hh = _silu(
        jnp.dot(f_ref[...], wh1_ref[0:ND, :], preferred_element_type=f32)
        + jnp.dot(agg, wh1_ref[ND:, :], preferred_element_type=f32)
        + bh1_ref[...]
    )
    h0_ref[...] = jnp.dot(hh, wh2_ref[...], preferred_element_type=f32) \
        + bh2_ref[...]
    v0_ref[...] = (p0_ref[1, 0] + p0_ref[1, 1]) + (p1_ref[1, 0] + p1_ref[1, 1])
    v1_ref[...] = (p0_ref[2, 0] + p0_ref[2, 1]) + (p1_ref[2, 0] + p1_ref[2, 1])
    v2_ref[...] = (p0_ref[3, 0] + p0_ref[3, 1]) + (p1_ref[3, 0] + p1_ref[3, 1])


def _final_tc(f, partials0, partials1, Wh1, bh1, Wh2, bh2):
    nb = lambda: pl.BlockSpec((BF, ND), lambda b: (b, 0))
    full = lambda *shape: pl.BlockSpec(shape, lambda b: (0,) * len(shape))
    ps = pl.BlockSpec((4, NC, BF, ND), lambda b: (0, 0, b, 0))
    fs = jax.ShapeDtypeStruct((N, ND), jnp.float32)
    return pl.pallas_call(
        _final_kernel,
        grid=(NF_BLOCKS,),
        in_specs=[
            nb(), ps, ps,
            full(2 * ND, ED), full(1, ED), full(ED, ND), full(1, ND),
        ],
        out_specs=[nb()] * 4,
        out_shape=[fs] * 4,
    )(f, partials0, partials1,
      Wh1, bh1.reshape(1, ED), Wh2, bh2.reshape(1, ND))


def kernel(species, edge_index, edge_attr, edge_vec,
           Wa, ba, Wb1, bb1, Wb2, bb2,
           Ws1, bs1, Ws2, bs2,
           Wh1, bh1, Wh2, bh2,
           Wv1, bv1, Wv2, bv2):
    i = edge_index[0]
    j = edge_index[1]

    f = _node_embed(species, Wa, ba)

    ea_h, part_h = [], []
    for h in range(2):
        sl = slice(h * EH, (h + 1) * EH)
        fi, fj = _gather_sc(f, i[sl], j[sl])
        ea, s, t0, t1, t2 = _edge_mlps(
            fi, fj, edge_attr[sl].T, edge_vec[sl].T,
            Wb1, bb1, Wb2, bb2, Ws1, bs1, Ws2, bs2, Wv1, bv1, Wv2, bv2)
        ea_h.append(ea)
        part_h.append(_scatter_sc(s, t0, t1, t2, j[sl]))

    h0, v00, v01, v02 = _final_tc(f, part_h[0], part_h[1],
                                  Wh1, bh1, Wh2, bh2)
    v0 = jnp.stack([v00, v01, v02], axis=-1)
    return (h0, v0, jnp.concatenate(ea_h, axis=0))


# speedup vs baseline: 31.3866x; 1.0291x over previous
import functools

import jax
import jax.numpy as jnp
from jax import lax
from jax.experimental import pallas as pl
from jax.experimental.pallas import tpu as pltpu
from jax.experimental.pallas import tpu_sc as plsc

N = 10000
E = 320000
ND = 128
IED = 16
ED = 128

NC = 2
NS = 16
NW = NC * NS

EH = E // 2

N_ACC = 10240

GCHUNK = 320
PER_TILE = 5120
GITERS = PER_TILE // GCHUNK
GITERS_LAST = (EH - 31 * PER_TILE) // GCHUNK

IW = 128
JROWS = EH // IW
JROWS_PAD = NW * (PER_TILE // IW)
TROWS = PER_TILE // IW
TROWS_LAST = JROWS - 31 * TROWS
NSLICE = N_ACC // NS

BD = 3200
ND_BLOCKS = EH // BD
BDE = 6400

BF = 1000
NF_BLOCKS = N // BF


def _silu(x):
    return x * jax.nn.sigmoid(x)


def _node_embed_kernel(species_ref, wa_ref, ba_ref, f_ref):
    f_ref[...] = (
        jnp.dot(species_ref[...], wa_ref[...],
                preferred_element_type=jnp.float32)
        + ba_ref[...]
    )


def _node_embed(species, Wa, ba):
    return pl.pallas_call(
        _node_embed_kernel,
        out_shape=jax.ShapeDtypeStruct((N, ND), jnp.float32),
    )(species, Wa, ba.reshape(1, ND))


def _gather_body(f_hbm, i_hbm, j_hbm, fi_hbm, fj_hbm,
                 idxi_v, idxj_v, rowsi_v, rowsj_v,
                 gsem, wsem_i, wsem_j):
    cid = lax.axis_index("c")
    sid = lax.axis_index("s")
    wid = cid * NS + sid
    base = wid * PER_TILE
    niter = jnp.where(wid == NW - 1, GITERS_LAST, GITERS)

    @pl.loop(0, niter)
    def _(k):
        off = base + k * GCHUNK
        pltpu.sync_copy(i_hbm.at[pl.ds(off, GCHUNK)], idxi_v)
        pltpu.sync_copy(j_hbm.at[pl.ds(off, GCHUNK)], idxj_v)

        @pl.when(k > 0)
        def _():
            pltpu.make_async_copy(rowsi_v, fi_hbm.at[pl.ds(off, GCHUNK)],
                                  wsem_i).wait()
        pltpu.async_copy(f_hbm.at[idxi_v], rowsi_v, gsem).wait()
        pltpu.async_copy(rowsi_v, fi_hbm.at[pl.ds(off, GCHUNK)], wsem_i)

        @pl.when(k > 0)
        def _():
            pltpu.make_async_copy(rowsj_v, fj_hbm.at[pl.ds(off, GCHUNK)],
                                  wsem_j).wait()
        pltpu.async_copy(f_hbm.at[idxj_v], rowsj_v, gsem).wait()
        pltpu.async_copy(rowsj_v, fj_hbm.at[pl.ds(off, GCHUNK)], wsem_j)

    pltpu.make_async_copy(rowsi_v, fi_hbm.at[pl.ds(base, GCHUNK)],
                          wsem_i).wait()
    pltpu.make_async_copy(rowsj_v, fj_hbm.at[pl.ds(base, GCHUNK)],
                          wsem_j).wait()


def _gather_sc(f, i, j):
    mesh = plsc.VectorSubcoreMesh(core_axis_name="c", subcore_axis_name="s")
    fs = jax.ShapeDtypeStruct((EH, ND), jnp.float32)
    run = pl.kernel(
        _gather_body,
        out_type=(fs, fs),
        mesh=mesh,
        scratch_types=[
            pltpu.VMEM((GCHUNK,), jnp.int32),
            pltpu.VMEM((GCHUNK,), jnp.int32),
            pltpu.VMEM((GCHUNK, ND), jnp.float32),
            pltpu.VMEM((GCHUNK, ND), jnp.float32),
            pltpu.SemaphoreType.DMA,
            pltpu.SemaphoreType.DMA,
            pltpu.SemaphoreType.DMA,
        ],
    )
    return run(f, i, j)


def _edge_ea_kernel(eattr_ref, wb1_ref, bb1_ref, wb2_ref, bb2_ref, ea_ref):
    f32 = jnp.float32
    eh = _silu(lax.dot_general(eattr_ref[...].astype(jnp.bfloat16),
                               wb1_ref[...], (((0,), (0,)), ((), ())),
                               preferred_element_type=f32) + bb1_ref[...])
    ea_ref[...] = jnp.dot(eh.astype(jnp.bfloat16), wb2_ref[...],
                          preferred_element_type=f32) + bb2_ref[...]


def _edge_ea(eattr_t, Wb1, bb1, Wb2, bb2):
    bf16 = jnp.bfloat16
    full = lambda *shape: pl.BlockSpec(shape, lambda b: (0,) * len(shape))
    return pl.pallas_call(
        _edge_ea_kernel,
        grid=(E // BDE,),
        in_specs=[
            pl.BlockSpec((IED, BDE), lambda b: (0, b)),
            full(IED, ED), full(1, ED), full(ED, ED), full(1, ED),
        ],
        out_specs=pl.BlockSpec((BDE, ND), lambda b: (b, 0)),
        out_shape=jax.ShapeDtypeStruct((E, ND), jnp.float32),
    )(eattr_t, Wb1.astype(bf16), bb1.reshape(1, ED),
      Wb2.astype(bf16), bb2.reshape(1, ED))


def _edge_mlp_kernel(fi_ref, fj_ref, ea_ref, evec_ref,
                     w1_ref, b1_ref, w2_ref, b2_ref,
                     s_ref, t0_ref, t1_ref, t2_ref):
    f32 = jnp.float32
    bf16 = jnp.bfloat16
    fi32 = fi_ref[...]
    fi = fi32.astype(bf16)
    fj = fj_ref[...].astype(bf16)

    h = _silu(
        jnp.dot(fi, w1_ref[0:ND, :], preferred_element_type=f32)
        + jnp.dot(fj, w1_ref[ND:2 * ND, :], preferred_element_type=f32)
        + jnp.dot(ea_ref[...].astype(bf16), w1_ref[2 * ND:, :],
                  preferred_element_type=f32)
        + b1_ref[...]
    ).astype(bf16)
    sp = jnp.dot(h, w2_ref[...], preferred_element_type=f32) + b2_ref[...]

    s_ref[...] = sp[:, 0:ND] * fi32
    pv = sp[:, ND:]
    ev = jnp.transpose(evec_ref[...])
    t0_ref[...] = pv * ev[:, 0:1]
    t1_ref[...] = pv * ev[:, 1:2]
    t2_ref[...] = pv * ev[:, 2:3]


def _edge_mlps(fi, fj, ea_full, evec_t, half,
               Ws1, bs1, Ws2, bs2, Wv1, bv1, Wv2, bv2):
    bf16 = jnp.bfloat16
    w1 = jnp.concatenate([Ws1, Wv1], axis=1).astype(bf16)
    b1 = jnp.concatenate([bs1, bv1]).reshape(1, 2 * ND)
    w2 = jnp.zeros((2 * ND, 2 * ND), jnp.float32)
    w2 = w2.at[0:ND, 0:ND].set(Ws2).at[ND:, ND:].set(Wv2).astype(bf16)
    b2 = jnp.concatenate([bs2, bv2]).reshape(1, 2 * ND)

    hoff = half * (EH // BD)
    eb = lambda: pl.BlockSpec((BD, ND), lambda b: (b, 0))
    full = lambda *shape: pl.BlockSpec(shape, lambda b: (0,) * len(shape))
    fs = jax.ShapeDtypeStruct((EH, ND), jnp.float32)
    return pl.pallas_call(
        _edge_mlp_kernel,
        grid=(ND_BLOCKS,),
        in_specs=[
            eb(), eb(),
            pl.BlockSpec((BD, ND), lambda b: (hoff + b, 0)),
            pl.BlockSpec((3, BD), lambda b: (0, b)),
            full(2 * ND + ED, 2 * ND), full(1, 2 * ND),
            full(2 * ND, 2 * ND), full(1, 2 * ND),
        ],
        out_specs=[eb()] * 4,
        out_shape=[fs] * 4,
    )(fi, fj, ea_full, evec_t, w1, b1, w2, b2)


def _scatter_body(s_hbm, t0_hbm, t1_hbm, t2_hbm, j2_hbm, z_hbm, out_hbm,
                  idx_v, vals0_v, vals1_v, acc, dsem0, dsem1):
    cid = lax.axis_index("c")
    sid = lax.axis_index("s")
    tile = cid * NS + sid
    row0 = tile * TROWS
    ebase = tile * PER_TILE
    pairs = jnp.where(tile == NW - 1, TROWS_LAST // 2, TROWS // 2)

    pltpu.sync_copy(j2_hbm.at[pl.ds(row0, TROWS)], idx_v)

    for p, vals_hbm in enumerate((s_hbm, t0_hbm, t1_hbm, t2_hbm)):
        pltpu.sync_copy(z_hbm.at[pl.ds(sid * NSLICE, NSLICE)],
                        acc.at[pl.ds(sid * NSLICE, NSLICE)])
        plsc.subcore_barrier()

        pltpu.async_copy(vals_hbm.at[pl.ds(ebase, IW)], vals0_v, dsem0)

        @pl.loop(0, pairs)
        def _(g):
            q0 = 2 * g
            pltpu.async_copy(vals_hbm.at[pl.ds(ebase + (q0 + 1) * IW, IW)],
                             vals1_v, dsem1)
            pltpu.make_async_copy(vals_hbm.at[pl.ds(ebase, IW)], vals0_v,
                                  dsem0).wait()
            pltpu.sync_copy(vals0_v, acc.at[idx_v.at[q0]], add=True)

            @pl.when(g < pairs - 1)
            def _():
                pltpu.async_copy(
                    vals_hbm.at[pl.ds(ebase + (q0 + 2) * IW, IW)],
                    vals0_v, dsem0)
            pltpu.make_async_copy(vals_hbm.at[pl.ds(ebase, IW)], vals1_v,
                                  dsem1).wait()
            pltpu.sync_copy(vals1_v, acc.at[idx_v.at[q0 + 1]], add=True)

        plsc.subcore_barrier()
        pltpu.sync_copy(acc.at[pl.ds(sid * NSLICE, NSLICE)],
                        out_hbm.at[p, cid, pl.ds(sid * NSLICE, NSLICE)])
        plsc.subcore_barrier()


def _scatter_sc(s, t0, t1, t2, j):
    mesh = plsc.VectorSubcoreMesh(core_axis_name="c", subcore_axis_name="s")
    j2 = jnp.concatenate(
        [j, jnp.zeros(JROWS_PAD * IW - EH, jnp.int32)]).reshape(JROWS_PAD, IW)
    zeros = jnp.zeros((N_ACC, ND), jnp.float32)
    run = pl.kernel(
        _scatter_body,
        out_type=jax.ShapeDtypeStruct((4, NC, N_ACC, ND), jnp.float32),
        mesh=mesh,
        scratch_types=[
            pltpu.VMEM((TROWS, IW), jnp.int32),
            pltpu.VMEM((IW, ND), jnp.float32),
            pltpu.VMEM((IW, ND), jnp.float32),
            pltpu.VMEM_SHARED((N_ACC, ND), jnp.float32),
            pltpu.SemaphoreType.DMA,
            pltpu.SemaphoreType.DMA,
        ],
    )
    return run(s, t0, t1, t2, j2, zeros)


def _final_kernel(f_ref, p0_ref, p1_ref, wh1_ref, bh1_ref, wh2_ref, bh2_ref,
                  h0_ref, v0_ref, v1_ref, v2_ref):
    f32 = jnp.float32
    agg = (p0_ref[0, 0] + p0_ref[0, 1]) + (p1_ref[0, 0] + p1_ref[0, 1])
    hh = _silu(
        jnp.dot(f_ref[...], wh1_ref[0:ND, :], preferred_element_type=f32)
        + jnp.dot(agg, wh1_ref[ND:, :], preferred_element_type=f32)
        + bh1_ref[...]
    )
    h0_ref[...] = jnp.dot(hh, wh2_ref[...], preferred_element_type=f32) \
        + bh2_ref[...]
    v0_ref[...] = (p0_ref[1, 0] + p0_ref[1, 1]) + (p1_ref[1, 0] + p1_ref[1, 1])
    v1_ref[...] = (p0_ref[2, 0] + p0_ref[2, 1]) + (p1_ref[2, 0] + p1_ref[2, 1])
    v2_ref[...] = (p0_ref[3, 0] + p0_ref[3, 1]) + (p1_ref[3, 0] + p1_ref[3, 1])


def _final_tc(f, partials0, partials1, Wh1, bh1, Wh2, bh2):
    nb = lambda: pl.BlockSpec((BF, ND), lambda b: (b, 0))
    full = lambda *shape: pl.BlockSpec(shape, lambda b: (0,) * len(shape))
    ps = pl.BlockSpec((4, NC, BF, ND), lambda b: (0, 0, b, 0))
    fs = jax.ShapeDtypeStruct((N, ND), jnp.float32)
    return pl.pallas_call(
        _final_kernel,
        grid=(NF_BLOCKS,),
        in_specs=[
            nb(), ps, ps,
            full(2 * ND, ED), full(1, ED), full(ED, ND), full(1, ND),
        ],
        out_specs=[nb()] * 4,
        out_shape=[fs] * 4,
    )(f, partials0, partials1,
      Wh1, bh1.reshape(1, ED), Wh2, bh2.reshape(1, ND))


def kernel(species, edge_index, edge_attr, edge_vec,
           Wa, ba, Wb1, bb1, Wb2, bb2,
           Ws1, bs1, Ws2, bs2,
           Wh1, bh1, Wh2, bh2,
           Wv1, bv1, Wv2, bv2):
    i = edge_index[0]
    j = edge_index[1]

    f = _node_embed(species, Wa, ba)
    ea = _edge_ea(edge_attr.T, Wb1, bb1, Wb2, bb2)

    part_h = []
    for h in range(2):
        sl = slice(h * EH, (h + 1) * EH)
        fi, fj = _gather_sc(f, i[sl], j[sl])
        s, t0, t1, t2 = _edge_mlps(
            fi, fj, ea, edge_vec[sl].T, h,
            Ws1, bs1, Ws2, bs2, Wv1, bv1, Wv2, bv2)
        part_h.append(_scatter_sc(s, t0, t1, t2, j[sl]))

    h0, v00, v01, v02 = _final_tc(f, part_h[0], part_h[1],
                                  Wh1, bh1, Wh2, bh2)
    v0 = jnp.stack([v00, v01, v02], axis=-1)
    return (h0, v0, ea)
